# baseline (speedup 1.0000x reference)
import functools

import jax
import jax.numpy as jnp
from jax import lax
from jax.experimental import pallas as pl
from jax.experimental.pallas import tpu as pltpu
from jax.experimental.pallas import tpu_sc as plsc

_N = 10000
_NACC = 10240
_TRASH = 10000
_C = 128
_NS = 16
_NC = 2
_MMBLK = 1000


def _mesh():
    return plsc.VectorSubcoreMesh(core_axis_name="c", subcore_axis_name="s")


def _writeout_rows(acc_sh, out_slice_fn, s):
    @pl.when(s < _NS - 1)
    def _():
        pltpu.sync_copy(acc_sh.at[pl.ds(s * 640, 640)], out_slice_fn(s * 640, 640))

    @pl.when(s == _NS - 1)
    def _():
        pltpu.sync_copy(acc_sh.at[pl.ds(9600, 400)], out_slice_fn(9600, 400))


_D = 2


def _sc_hop(tbl, src_i, dst2d, w3d=None, width=128, edge_split=False):
    epr = dst2d.shape[0]
    k = epr // (_NC * _NS) if edge_split else epr // _NS
    nq = k // 8
    assert k % 8 == 0 and nq >= 2
    weighted = w3d is not None
    nvec = width // 16

    def body(*refs):
        (t_hbm, s_hbm, d_hbm, w_hbm, out_hbm) = refs[:5]
        rest = list(refs[5:])
        sig = rest.pop(0)
        dig = rest.pop(0)
        sif = [rest.pop(0) for _ in range(2)]
        dif = [rest.pop(0) for _ in range(2)]
        rows = [rest.pop(0) for _ in range(2)]
        w_v = [rest.pop(0) for _ in range(2)] if weighted else None
        acc_sh = rest.pop(0)
        gsem = [rest.pop(0) for _ in range(2)]
        isem = rest.pop(0)
        wsem = [rest.pop(0) for _ in range(2)] if weighted else None

        c = lax.axis_index("c")
        s = lax.axis_index("s")
        t0 = ((c * _NS + s) if edge_split else s) * k

        def zrow(r, rc):
            zv = jnp.zeros((16,), jnp.float32)
            for u in range(nvec):
                rows[0][r, pl.ds(u * 16, 16)] = zv
            return rc
        lax.fori_loop(0, _C, zrow, 0)
        zr = _NACC // _NS
        for zi in range(zr // _C):
            pltpu.sync_copy(rows[0], acc_sh.at[pl.ds(s * zr + zi * _C, _C)])
        plsc.subcore_barrier()

        def src_grp(q):
            if edge_split:
                return s_hbm.at[pl.ds(t0 + q * 8, 8), :]
            return s_hbm.at[c, pl.ds(t0 + q * 8, 8), :]

        def issue_idx(q, qb):
            pltpu.async_copy(src_grp(q), sig.at[qb], isem)
            pltpu.async_copy(d_hbm.at[pl.ds(t0 + q * 8, 8), :], dig.at[qb], isem)

        def wait_idx(qb):
            pltpu.make_async_copy(src_grp(0), sig.at[qb], isem).wait()
            pltpu.make_async_copy(d_hbm.at[pl.ds(t0, 8), :], dig.at[qb], isem).wait()

        def issue_gather(qb, r8, b):
            for u in range(8):
                sif[b][pl.ds(u * 16, 16)] = sig[qb, r8, pl.ds(u * 16, 16)]
            pltpu.async_copy(t_hbm.at[sif[b]], rows[b], gsem[b])

        def issue_w(j, b):
            if weighted:
                pltpu.async_copy(w_hbm.at[t0 + j], w_v[b], wsem[b])

        def wait_gather(b):
            pltpu.make_async_copy(t_hbm.at[sif[b]], rows[b], gsem[b]).wait()
            if weighted:
                pltpu.make_async_copy(w_hbm.at[0], w_v[b], wsem[b]).wait()

        def multiply(b):
            if not weighted:
                return

            def row(r, rc):
                wv = w_v[b][r, :]
                for u in range(nvec):
                    rows[b][r, pl.ds(u * 16, 16)] = rows[b][r, pl.ds(u * 16, 16)] * wv
                return rc
            lax.fori_loop(0, _C, row, 0)

        def group_slots(q, qp, last_group):
            for r8 in range(8):
                j = q * 8 + r8
                b = r8 % 2
                wait_gather(b)
                multiply(b)
                if r8 < 7:
                    issue_gather(qp, r8 + 1, 1 - b)
                    issue_w(j + 1, 1 - b)
                elif not last_group:
                    wait_idx(1 - qp)
                    issue_gather(1 - qp, 0, 1 - b)
                    issue_w(j + 1, 1 - b)
                for u in range(8):
                    dif[b][pl.ds(u * 16, 16)] = dig[qp, r8, pl.ds(u * 16, 16)]
                pltpu.sync_copy(rows[b], acc_sh.at[dif[b]], add=True)

        issue_idx(0, 0)
        wait_idx(0)
        issue_gather(0, 0, 0)
        issue_w(0, 0)

        def group(q, carry):
            qp = lax.rem(q, 2)
            issue_idx(q + 1, 1 - qp)
            group_slots(q, qp, False)
            return carry
        lax.fori_loop(0, nq - 1, group, 0)
        group_slots(nq - 1, (nq - 1) % 2, True)

        plsc.subcore_barrier()
        _writeout_rows(acc_sh, lambda r0, nr: out_hbm.at[c, pl.ds(r0, nr), :], s)

    scratch = [pltpu.VMEM((2, 8, _C), jnp.int32) for _ in range(2)]
    scratch += [pltpu.VMEM((_C,), jnp.int32) for _ in range(4)]
    scratch += [pltpu.VMEM((_C, width), jnp.float32) for _ in range(2)]
    if weighted:
        scratch += [pltpu.VMEM((_C, 16), jnp.float32) for _ in range(2)]
    scratch.append(pltpu.VMEM_SHARED((_NACC, width), jnp.float32))
    scratch += [pltpu.SemaphoreType.DMA for _ in range(3)]
    if weighted:
        scratch += [pltpu.SemaphoreType.DMA for _ in range(2)]

    f = pl.kernel(
        body,
        out_type=jax.ShapeDtypeStruct((2, _N, width), jnp.float32),
        mesh=_mesh(),
        scratch_types=scratch,
        compiler_params=pltpu.CompilerParams(use_tc_tiling_on_sc=False),
    )
    warg = w3d if weighted else jnp.zeros((1, 16), jnp.float32)
    return f(tbl, src_i, dst2d, warg)


def _sc_scatter16(v3d, dst2d):
    epr = dst2d.shape[0]
    k = epr // (_NC * _NS)
    kq = k // _D
    assert k % _D == 0 and kq >= 2

    def body(v_hbm, d_hbm, z_hbm, out_hbm, *rest):
        rest = list(rest)
        di_all = rest.pop(0)
        dif = [rest.pop(0) for _ in range(_D)]
        rows = [rest.pop(0) for _ in range(_D)]
        acc_sh = rest.pop(0)
        gsem = [rest.pop(0) for _ in range(_D)]
        ssem = [rest.pop(0) for _ in range(_D)]

        c = lax.axis_index("c")
        s = lax.axis_index("s")
        t0 = (c * _NS + s) * k
        pltpu.sync_copy(d_hbm.at[pl.ds(t0, k), :], di_all)
        zr = _NACC // _NS
        pltpu.sync_copy(z_hbm.at[pl.ds(s * zr, zr)], acc_sh.at[pl.ds(s * zr, zr)])
        plsc.subcore_barrier()

        def issue_gather(j, b):
            pltpu.async_copy(v_hbm.at[t0 + j], rows[b], gsem[b])

        def wait_gather(b):
            pltpu.make_async_copy(v_hbm.at[0], rows[b], gsem[b]).wait()

        def issue_scatter(j, b):
            for u in range(8):
                dif[b][pl.ds(u * 16, 16)] = di_all[j, pl.ds(u * 16, 16)]
            pltpu.async_copy(rows[b], acc_sh.at[dif[b]], ssem[b], add=True)

        def wait_scatter(b):
            pltpu.make_async_copy(rows[b], acc_sh.at[dif[b]], ssem[b]).wait()

        def slot(j, b, first, issue_next):
            wait_gather(b)
            issue_scatter(j, b)
            if not first and issue_next:
                bp = (b - 1) % _D
                wait_scatter(bp)
                issue_gather(j + _D - 1, bp)

        for b in range(_D):
            issue_gather(b, b)
        for b in range(_D):
            slot(b, b, b == 0, True)

        def group(g, carry):
            for b in range(_D):
                slot(g * _D + b, b, False, True)
            return carry
        lax.fori_loop(1, kq - 1, group, 0)

        for b in range(_D):
            slot((kq - 1) * _D + b, b, False, b == 0)
        for b in range(_D):
            wait_scatter(b)

        plsc.subcore_barrier()
        _writeout_rows(acc_sh, lambda r0, nr: out_hbm.at[c, pl.ds(r0, nr), :], s)

    scratch = [pltpu.VMEM((k, 128), jnp.int32)]
    scratch += [pltpu.VMEM((_C,), jnp.int32) for _ in range(_D)]
    scratch += [pltpu.VMEM((_C, 16), jnp.float32) for _ in range(_D)]
    scratch.append(pltpu.VMEM_SHARED((_NACC, 16), jnp.float32))
    scratch += [pltpu.SemaphoreType.DMA for _ in range(2 * _D)]

    zeros = jnp.zeros((_NACC, 16), jnp.float32)
    f = pl.kernel(
        body,
        out_type=jax.ShapeDtypeStruct((2, _N, 16), jnp.float32),
        mesh=_mesh(),
        scratch_types=scratch,
    )
    return f(v3d, dst2d, zeros)


def _sc_gather_pair(ta, tb, ia_i, ib_i, width=128, edge_split=False):
    dp = 2
    epr = ia_i.shape[-2]
    ep = epr * _C
    k = epr // (_NC * _NS) if edge_split else epr // _NS
    kq = k // dp
    assert k % dp == 0 and kq >= 2

    def body(ta_hbm, tb_hbm, ia_hbm, ib_hbm, oa_hbm, ob_hbm, *rest):
        rest = list(rest)
        ia_all = rest.pop(0)
        ib_all = rest.pop(0)
        ra = [rest.pop(0) for _ in range(dp)]
        rb = [rest.pop(0) for _ in range(dp)]
        gsa = [rest.pop(0) for _ in range(dp)]
        gsb = [rest.pop(0) for _ in range(dp)]
        wsa = [rest.pop(0) for _ in range(dp)]
        wsb = [rest.pop(0) for _ in range(dp)]

        c = lax.axis_index("c")
        s = lax.axis_index("s")
        t0 = ((c * _NS + s) if edge_split else s) * k
        if edge_split:
            pltpu.sync_copy(ia_hbm.at[pl.ds(t0, k), :], ia_all)
            pltpu.sync_copy(ib_hbm.at[pl.ds(t0, k), :], ib_all)
        else:
            pltpu.sync_copy(ia_hbm.at[c, pl.ds(t0, k), :], ia_all)
            pltpu.sync_copy(ib_hbm.at[c, pl.ds(t0, k), :], ib_all)

        def out_a(j):
            if edge_split:
                return oa_hbm.at[pl.ds((t0 + j) * _C, _C), :]
            return oa_hbm.at[c, pl.ds((t0 + j) * _C, _C), :]

        def out_b(j):
            if edge_split:
                return ob_hbm.at[pl.ds((t0 + j) * _C, _C), :]
            return ob_hbm.at[c, pl.ds((t0 + j) * _C, _C), :]

        def issue_gather(j, b):
            pltpu.async_copy(ta_hbm.at[ia_all.at[j]], ra[b], gsa[b])
            pltpu.async_copy(tb_hbm.at[ib_all.at[j]], rb[b], gsb[b])

        def wait_gather(b):
            pltpu.make_async_copy(ta_hbm.at[ia_all.at[0]], ra[b], gsa[b]).wait()
            pltpu.make_async_copy(tb_hbm.at[ib_all.at[0]], rb[b], gsb[b]).wait()

        def issue_write(j, b):
            pltpu.async_copy(ra[b], out_a(j), wsa[b])
            pltpu.async_copy(rb[b], out_b(j), wsb[b])

        def wait_write(b):
            pltpu.make_async_copy(ra[b], out_a(0), wsa[b]).wait()
            pltpu.make_async_copy(rb[b], out_b(0), wsb[b]).wait()

        def slot(j, b, first, issue_next):
            wait_gather(b)
            issue_write(j, b)
            if not first and issue_next:
                bp = (b - 1) % dp
                wait_write(bp)
                issue_gather(j + dp - 1, bp)

        for b in range(dp):
            issue_gather(b, b)
        for b in range(dp):
            slot(b, b, b == 0, True)

        def group(g, carry):
            for b in range(dp):
                slot(g * dp + b, b, False, True)
            return carry
        lax.fori_loop(1, kq - 1, group, 0)

        for b in range(dp):
            slot((kq - 1) * dp + b, b, False, b == 0)
        for b in range(dp):
            wait_write(b)

    scratch = [
        pltpu.VMEM((k, 128), jnp.int32),
        pltpu.VMEM((k, 128), jnp.int32),
    ]
    scratch += [pltpu.VMEM((_C, width), jnp.float32) for _ in range(2 * dp)]
    scratch += [pltpu.SemaphoreType.DMA for _ in range(4 * dp)]

    if edge_split:
        out_sd = jax.ShapeDtypeStruct((ep, width), jnp.float32)
    else:
        out_sd = jax.ShapeDtypeStruct((2, ep, width), jnp.float32)
    params = {}
    if width == 16:
        params["compiler_params"] = pltpu.CompilerParams(use_tc_tiling_on_sc=False)
    f = pl.kernel(
        body,
        out_type=(out_sd, out_sd),
        mesh=_mesh(),
        scratch_types=scratch,
        **params,
    )
    return f(ta, tb, ia_i, ib_i)


def _mm_h(xh, w, bias=None, relu=False, acc=None, row_scale=None):
    nb = _N // _MMBLK
    has_b = bias is not None
    has_a = acc is not None
    has_s = row_scale is not None

    def body(*refs):
        i = 0
        x_ref = refs[i]; i += 1
        w_ref = refs[i]; i += 1
        s_ref = refs[i] if has_s else None
        i += has_s
        b_ref = refs[i] if has_b else None
        i += has_b
        a_ref = refs[i] if has_a else None
        i += has_a
        o_ref = refs[i]
        ci = pl.program_id(2)
        xv = x_ref[...]
        if has_s:
            xv = xv * s_ref[...]
        contrib = jnp.dot(xv, w_ref[...], preferred_element_type=jnp.float32)

        @pl.when(ci == 0)
        def _():
            r = contrib
            if has_b:
                r = r + b_ref[...]
            if has_a:
                r = r + a_ref[...]
            o_ref[...] = r

        @pl.when(ci == 1)
        def _():
            r = o_ref[...] + contrib
            if relu:
                r = jnp.maximum(r, 0.0)
            o_ref[...] = r

    in_specs = [
        pl.BlockSpec((_MMBLK, 128), lambda i, co, ci: (ci * nb + i, 0)),
        pl.BlockSpec((128, 128), lambda i, co, ci: (ci, co)),
    ]
    args = [xh, w]
    if has_s:
        in_specs.append(pl.BlockSpec((_MMBLK, 1), lambda i, co, ci: (ci * nb + i, 0)))
        args.append(row_scale)
    if has_b:
        in_specs.append(pl.BlockSpec((1, 128), lambda i, co, ci: (0, co)))
        args.append(bias.reshape(1, 256))
    if has_a:
        in_specs.append(pl.BlockSpec((_MMBLK, 128), lambda i, co, ci: (co * nb + i, 0)))
        args.append(acc)
    return pl.pallas_call(
        body,
        grid=(nb, 2, 2),
        in_specs=in_specs,
        out_specs=pl.BlockSpec((_MMBLK, 128), lambda i, co, ci: (co * nb + i, 0)),
        out_shape=jax.ShapeDtypeStruct((2 * _N, 128), jnp.float32),
    )(*args)


def _mm_thin(xh, w16):
    nb = _N // _MMBLK

    def body(x0_ref, x1_ref, w_ref, o_ref):
        o_ref[...] = (
            jnp.dot(x0_ref[...], w_ref[0:128, :], preferred_element_type=jnp.float32)
            + jnp.dot(x1_ref[...], w_ref[128:256, :], preferred_element_type=jnp.float32))

    return pl.pallas_call(
        body,
        grid=(nb,),
        in_specs=[
            pl.BlockSpec((_MMBLK, 128), lambda i: (i, 0)),
            pl.BlockSpec((_MMBLK, 128), lambda i: (nb + i, 0)),
            pl.BlockSpec((256, 16), lambda i: (0, 0)),
        ],
        out_specs=pl.BlockSpec((_MMBLK, 16), lambda i: (i, 0)),
        out_shape=jax.ShapeDtypeStruct((_N, 16), jnp.float32),
    )(xh, xh, w16)


_EBLK = 1024


def _tc_score128(gl, gr, att):
    ep = gl.shape[1]
    ne = ep // _EBLK

    def body(gl0, gl1, gr0, gr1, att_ref, s_ref, m_ref):
        i = pl.program_id(0)
        t0 = gl0[0] + gr0[0]
        t1 = gl1[0] + gr1[0]
        t0 = jnp.where(t0 >= 0, t0, 0.2 * t0)
        t1 = jnp.where(t1 >= 0, t1, 0.2 * t1)
        s = (jnp.sum(t0 * att_ref[0:1, :], axis=-1, keepdims=True)
             + jnp.sum(t1 * att_ref[1:2, :], axis=-1, keepdims=True))
        s_ref[...] = s
        bm = jnp.max(s, keepdims=True)

        @pl.when(i == 0)
        def _():
            m_ref[...] = bm

        @pl.when(i > 0)
        def _():
            m_ref[...] = jnp.maximum(m_ref[...], bm)

    return pl.pallas_call(
        body,
        grid=(ne,),
        in_specs=[
            pl.BlockSpec((1, _EBLK, 128), lambda i: (0, i, 0)),
            pl.BlockSpec((1, _EBLK, 128), lambda i: (1, i, 0)),
            pl.BlockSpec((1, _EBLK, 128), lambda i: (0, i, 0)),
            pl.BlockSpec((1, _EBLK, 128), lambda i: (1, i, 0)),
            pl.BlockSpec((2, 128), lambda i: (0, 0)),
        ],
        out_specs=[
            pl.BlockSpec((_EBLK, 1), lambda i: (i, 0)),
            pl.BlockSpec((1, 1), lambda i: (0, 0)),
        ],
        out_shape=[
            jax.ShapeDtypeStruct((ep, 1), jnp.float32),
            jax.ShapeDtypeStruct((1, 1), jnp.float32),
        ],
    )(gl, gl, gr, gr, att.reshape(2, 128))


def _tc_score16(g1, g2, att0):
    ep = g1.shape[0]
    ne = ep // _EBLK

    def body(g1_ref, g2_ref, a_ref, s_ref, m_ref):
        i = pl.program_id(0)
        t = g1_ref[:, 0:1] + g2_ref[:, 1:2]
        t = jnp.where(t >= 0, t, 0.2 * t)
        s = t * a_ref[0, 0]
        s_ref[...] = s
        bm = jnp.max(s, keepdims=True)

        @pl.when(i == 0)
        def _():
            m_ref[...] = bm

        @pl.when(i > 0)
        def _():
            m_ref[...] = jnp.maximum(m_ref[...], bm)

    return pl.pallas_call(
        body,
        grid=(ne,),
        in_specs=[
            pl.BlockSpec((_EBLK, 16), lambda i: (i, 0)),
            pl.BlockSpec((_EBLK, 16), lambda i: (i, 0)),
            pl.BlockSpec((1, 1), lambda i: (0, 0)),
        ],
        out_specs=[
            pl.BlockSpec((_EBLK, 1), lambda i: (i, 0)),
            pl.BlockSpec((1, 1), lambda i: (0, 0)),
        ],
        out_shape=[
            jax.ShapeDtypeStruct((ep, 1), jnp.float32),
            jax.ShapeDtypeStruct((1, 1), jnp.float32),
        ],
    )(g1, g2, att0.reshape(1, 1))


def _tc_exp16(s, gmax):
    ep = s.shape[0]
    ne = ep // _EBLK

    def body(s_ref, m_ref, o_ref):
        e = jnp.exp(s_ref[...] - m_ref[0, 0])
        o_ref[...] = jnp.broadcast_to(e, (_EBLK, 16))

    return pl.pallas_call(
        body,
        grid=(ne,),
        in_specs=[
            pl.BlockSpec((_EBLK, 1), lambda i: (i, 0)),
            pl.BlockSpec((1, 1), lambda i: (0, 0)),
        ],
        out_specs=pl.BlockSpec((_EBLK, 16), lambda i: (i, 0)),
        out_shape=jax.ShapeDtypeStruct((ep, 16), jnp.float32),
    )(s, gmax)


def _tc_rowscale(a, s2, bias=None, relu=False, div=False):
    nb = _N // _MMBLK
    has_b = bias is not None

    def body(*refs):
        a_ref, s_ref = refs[0], refs[1]
        b_ref = refs[2] if has_b else None
        o_ref = refs[-1]
        v = a_ref[...] / s_ref[...] if div else a_ref[...] * s_ref[...]
        if has_b:
            v = v + b_ref[...]
        if relu:
            v = jnp.maximum(v, 0.0)
        o_ref[...] = v

    in_specs = [
        pl.BlockSpec((_MMBLK, 128), lambda i: (i, 0)),
        pl.BlockSpec((_MMBLK, 1), lambda i: (i, 0)),
    ]
    args = [a, s2]
    if has_b:
        in_specs.append(pl.BlockSpec((1, 128), lambda i: (0, i // nb)))
        args.append(bias.reshape(1, 256))
    return pl.pallas_call(
        body,
        grid=(2 * nb,),
        in_specs=in_specs,
        out_specs=pl.BlockSpec((_MMBLK, 128), lambda i: (i, 0)),
        out_shape=jax.ShapeDtypeStruct((2 * _N, 128), jnp.float32),
    )(*args)


def _tc_scale16(a, s):
    nb = _N // _MMBLK

    def body(a_ref, s_ref, o_ref):
        o_ref[...] = a_ref[...] * s_ref[...]

    return pl.pallas_call(
        body,
        grid=(nb,),
        in_specs=[
            pl.BlockSpec((_MMBLK, 16), lambda i: (i, 0)),
            pl.BlockSpec((_MMBLK, 1), lambda i: (i, 0)),
        ],
        out_specs=pl.BlockSpec((_MMBLK, 16), lambda i: (i, 0)),
        out_shape=jax.ShapeDtypeStruct((_N, 16), jnp.float32),
    )(a, s)


def _tc_scale16p(p, s):
    nb = _N // _MMBLK

    def body(p_ref, s_ref, o_ref):
        o_ref[...] = (p_ref[0] + p_ref[1]) * s_ref[...]

    return pl.pallas_call(
        body,
        grid=(nb,),
        in_specs=[
            pl.BlockSpec((2, _MMBLK, 16), lambda i: (0, i, 0)),
            pl.BlockSpec((_MMBLK, 1), lambda i: (i, 0)),
        ],
        out_specs=pl.BlockSpec((_MMBLK, 16), lambda i: (i, 0)),
        out_shape=jax.ShapeDtypeStruct((_N, 16), jnp.float32),
    )(p, s)


def _tc_prep(degp):
    nb = _N // _MMBLK

    def body(d_ref, dis_ref, ic_ref):
        deg = d_ref[0, :, 0:1] + d_ref[1, :, 0:1]
        dis = jnp.where(deg > 0, jax.lax.rsqrt(jnp.maximum(deg, 1e-12)), 0.0)
        dis_ref[...] = dis
        ic_ref[...] = 1.0 / jnp.maximum(deg, 1.0)

    return pl.pallas_call(
        body,
        grid=(nb,),
        in_specs=[pl.BlockSpec((2, _MMBLK, 16), lambda i: (0, i, 0))],
        out_specs=[
            pl.BlockSpec((_MMBLK, 1), lambda i: (i, 0)),
            pl.BlockSpec((_MMBLK, 1), lambda i: (i, 0)),
        ],
        out_shape=[
            jax.ShapeDtypeStruct((_N, 1), jnp.float32),
            jax.ShapeDtypeStruct((_N, 1), jnp.float32),
        ],
    )(degp)


def _tc_final(n4p, z4p, u_tag, p1, p2, p3, v_sage, hs_p, invcnt, scal, y, mask):
    nb = _N // _MMBLK

    def body(n4, z4, ut, p1r, p2r, p3r, vs, hs, ic, sc, y_ref, m_ref, o_ref):
        b4 = sc[0, 0]
        btag = sc[1, 0]
        bl3 = sc[2, 0]
        w0, w1, w2, blin = sc[3, 0], sc[4, 0], sc[5, 0], sc[6, 0]
        x1 = (n4[0, :, 0:1] + n4[1, :, 0:1]) / (z4[0, :, 0:1] + z4[1, :, 0:1]) + b4
        x2 = ut[:, 3:4] + p1r[:, 0:1] + p2r[:, 1:2] + p3r[:, 2:3] + btag
        x3 = (hs[0, :, 0:1] + hs[1, :, 0:1]) * ic[...] + bl3 + vs[:, 1:2]
        out = jnp.maximum(x1 * w0 + x2 * w1 + x3 * w2 + blin, 0.0)
        x_i = jnp.where(m_ref[...] != 0, out / 0.05, 0.0)
        o_ref[...] = jnp.where(y_ref[...] == 0.0, x_i, out)

    blk2 = pl.BlockSpec((2, _MMBLK, 16), lambda i: (0, i, 0))
    blk16 = pl.BlockSpec((_MMBLK, 16), lambda i: (i, 0))
    blk1 = pl.BlockSpec((_MMBLK, 1), lambda i: (i, 0))
    return pl.pallas_call(
        body,
        grid=(nb,),
        in_specs=[blk2, blk2, blk16, blk16, blk16, blk16, blk16, blk2,
                  blk1, pl.BlockSpec((7, 1), lambda i: (0, 0)), blk1, blk1],
        out_specs=blk1,
        out_shape=jax.ShapeDtypeStruct((_N, 1), jnp.float32),
    )(n4p, z4p, u_tag, p1, p2, p3, v_sage, hs_p, invcnt, scal, y, mask)


def _pad1(a, ep, fill=0):
    return jnp.concatenate(
        [a.astype(jnp.int32),
         jnp.full((ep - a.shape[0],), fill, jnp.int32)])


def _pad_trash(a, ep):
    npad = ep - a.shape[0]
    fill = _TRASH + (jnp.arange(npad, dtype=jnp.int32) % (_NACC - _N))
    return jnp.concatenate([a.astype(jnp.int32), fill])


def kernel(x, edge_index, y, params):
    n = _N
    src = edge_index[0].astype(jnp.int32)
    dst = edge_index[1].astype(jnp.int32)
    e = src.shape[0]
    loop = jnp.arange(n, dtype=jnp.int32)
    gran = _NC * _NS * _C * 8

    ep1 = ((e + gran - 1) // gran) * gran
    ep2 = ((e + n + gran - 1) // gran) * gran
    epr1, epr2 = ep1 // _C, ep2 // _C

    srcp1 = _pad1(src, ep1)
    dstp1 = _pad_trash(dst, ep1)
    src1_2d = srcp1.reshape(epr1, _C)
    dst1_2d = dstp1.reshape(epr1, _C)
    src2_1 = jnp.stack([srcp1, srcp1 + n]).reshape(2, epr1, _C)

    src_sl = jnp.concatenate([src, loop])
    dst_sl = jnp.concatenate([dst, loop])
    srcp2 = _pad1(src_sl, ep2)
    dstp2 = _pad_trash(dst_sl, ep2)
    dstg2 = _pad1(dst_sl, ep2)
    src2_2d = srcp2.reshape(epr2, _C)
    dst2_2d = dstp2.reshape(epr2, _C)
    dstg2_2d = dstg2.reshape(epr2, _C)
    src2_2 = jnp.stack([srcp2, srcp2 + n]).reshape(2, epr2, _C)
    dst2_2 = jnp.stack([dstg2, dstg2 + n]).reshape(2, epr2, _C)

    xh = jnp.transpose(x.reshape(n, 2, 128), (1, 0, 2)).reshape(2 * n, 128)

    ones3d = jnp.ones((epr1, _C, 16), jnp.float32)
    degp = _sc_scatter16(ones3d, dst1_2d)
    dis, invcnt = _tc_prep(degp)
    dis2 = jnp.concatenate([dis, dis], axis=0)
    dis2sq = dis2 * dis2
    invcnt2 = jnp.concatenate([invcnt, invcnt], axis=0)

    x1h = xh
    for p in params['gat'][:3]:
        xl = _mm_h(x1h, p['Wl'])
        xr = _mm_h(x1h, p['Wr'])
        gl, gr = _sc_gather_pair(xl, xr, src2_2, dst2_2, width=128)
        s, gmax = _tc_score128(gl, gr, p['att'])
        e16 = _tc_exp16(s, gmax)
        e3d = e16.reshape(epr2, _C, 16)
        zp = _sc_scatter16(e3d, dst2_2d)
        z = zp[0, :, 0:1] + zp[1, :, 0:1]
        z2 = jnp.concatenate([z, z], axis=0)
        numer = _sc_hop(xl, src2_2, dst2_2d, w3d=e3d, width=128).reshape(2 * n, 128)
        x1h = _tc_rowscale(numer, z2, bias=p['b'], relu=True, div=True)

    p4 = params['gat'][3]
    w4 = jnp.concatenate(
        [p4['Wl'], p4['Wr'], jnp.zeros((256, 14), jnp.float32)], axis=1)
    t4 = _mm_thin(x1h, w4)
    g1, g2 = _sc_gather_pair(t4, t4, src2_2d, dstg2_2d, width=16, edge_split=True)
    s4, gmax4 = _tc_score16(g1, g2, p4['att'])
    e4 = _tc_exp16(s4, gmax4)
    e4_3d = e4.reshape(epr2, _C, 16)
    z4p = _sc_scatter16(e4_3d, dst2_2d)
    n4p = _sc_hop(t4, src2_2d, dst2_2d, w3d=e4_3d, width=16, edge_split=True)

    x2h = xh
    for li, p in enumerate(params['tag'][:2]):
        out = _mm_h(x2h, p['Ws'][0])
        hs = _tc_rowscale(x2h, dis2)
        for kk in range(1, 4):
            raw = _sc_hop(hs, src2_1, dst1_2d, width=128).reshape(2 * n, 128)
            last = kk == 3
            out = _mm_h(raw, p['Ws'][kk], row_scale=dis2, acc=out,
                        bias=p['b'] if last else None, relu=last)
            if not last:
                hs = _tc_rowscale(raw, dis2sq)
        x2h = out

    p3t = params['tag'][2]
    w16t = jnp.concatenate(
        [p3t['Ws'][1], p3t['Ws'][2], p3t['Ws'][3], p3t['Ws'][0],
         jnp.zeros((256, 12), jnp.float32)], axis=1)
    u_tag = _mm_thin(x2h, w16t)
    q = _tc_scale16(u_tag, dis)
    h1 = _sc_hop(q, src1_2d, dst1_2d, width=16, edge_split=True)
    pp1 = _tc_scale16p(h1, dis)
    q = _tc_scale16(pp1, dis)
    h2 = _sc_hop(q, src1_2d, dst1_2d, width=16, edge_split=True)
    pp2 = _tc_scale16p(h2, dis)
    q = _tc_scale16(pp2, dis)
    h3 = _sc_hop(q, src1_2d, dst1_2d, width=16, edge_split=True)
    pp3 = _tc_scale16p(h3, dis)

    x3h = xh
    for p in params['sage'][:2]:
        raw = _sc_hop(x3h, src2_1, dst1_2d, width=128).reshape(2 * n, 128)
        out = _mm_h(raw, p['Wl'], row_scale=invcnt2, bias=p['bl'])
        x3h = _mm_h(x3h, p['Wr'], acc=out, relu=True)

    p3s = params['sage'][2]
    w16s = jnp.concatenate(
        [p3s['Wl'], p3s['Wr'], jnp.zeros((256, 14), jnp.float32)], axis=1)
    v_sage = _mm_thin(x3h, w16s)
    hs_p = _sc_hop(v_sage, src1_2d, dst1_2d, width=16, edge_split=True)

    scal = jnp.stack([
        p4['b'][0], p3t['b'][0], p3s['bl'][0],
        params['lin']['W'][0, 0], params['lin']['W'][1, 0],
        params['lin']['W'][2, 0], params['lin']['b'][0],
    ]).reshape(7, 1)
    mask = jax.random.bernoulli(jax.random.key(42), 0.05, (n, 1)).astype(jnp.float32)
    return _tc_final(n4p, z4p, u_tag, pp1, pp2, pp3, v_sage, hs_p, invcnt,
                     scal, y, mask)


# speedup vs baseline: 1.2166x; 1.2166x over previous
import functools

import jax
import jax.numpy as jnp
from jax import lax
from jax.experimental import pallas as pl
from jax.experimental.pallas import tpu as pltpu
from jax.experimental.pallas import tpu_sc as plsc

_N = 10000
_NACC = 10240
_TRASH = 10000
_C = 128
_NS = 16
_NC = 2
_MMBLK = 1000


def _mesh():
    return plsc.VectorSubcoreMesh(core_axis_name="c", subcore_axis_name="s")


def _writeout_rows(acc_sh, out_slice_fn, s):
    @pl.when(s < _NS - 1)
    def _():
        pltpu.sync_copy(acc_sh.at[pl.ds(s * 640, 640)], out_slice_fn(s * 640, 640))

    @pl.when(s == _NS - 1)
    def _():
        pltpu.sync_copy(acc_sh.at[pl.ds(9600, 400)], out_slice_fn(9600, 400))


_D = 2


def _sc_hop(tbl, src_i, dst, w2d=None, width=128, edge_split=False):
    cc = 64 if width == 128 else _C
    ep = dst.shape[0]
    k = ep // (_NC * _NS * cc) if edge_split else ep // (_NS * cc)
    kq = k // 2
    assert k % 2 == 0 and kq >= 2
    weighted = w2d is not None
    nvec = width // 16

    def body(*refs):
        (t_hbm, s_hbm, d_hbm, w_hbm, out_hbm) = refs[:5]
        rest = list(refs[5:])
        si = [rest.pop(0) for _ in range(2)]
        di = [rest.pop(0) for _ in range(2)]
        rows = [rest.pop(0) for _ in range(2)]
        w_v = [rest.pop(0) for _ in range(2)] if weighted else None
        acc_sh = rest.pop(0)
        gsem = [rest.pop(0) for _ in range(2)]
        wsem = [rest.pop(0) for _ in range(2)] if weighted else None

        c = lax.axis_index("c")
        s = lax.axis_index("s")
        t0 = ((c * _NS + s) if edge_split else s) * (k * cc)

        def zrow(r, rc):
            zv = jnp.zeros((16,), jnp.float32)
            for u in range(nvec):
                rows[0][r, pl.ds(u * 16, 16)] = zv
            return rc
        lax.fori_loop(0, cc, zrow, 0)
        zr = _NACC // _NS
        for zi in range(zr // cc):
            pltpu.sync_copy(rows[0], acc_sh.at[pl.ds(s * zr + zi * cc, cc)])
        plsc.subcore_barrier()

        def load_idx(j, b):
            if edge_split:
                pltpu.sync_copy(s_hbm.at[pl.ds(t0 + j * cc, cc)], si[b])
            else:
                pltpu.sync_copy(s_hbm.at[c, pl.ds(t0 + j * cc, cc)], si[b])
            pltpu.sync_copy(d_hbm.at[pl.ds(t0 + j * cc, cc)], di[b])

        def issue_gather(j, b):
            pltpu.async_copy(t_hbm.at[si[b]], rows[b], gsem[b])
            if weighted:
                pltpu.async_copy(w_hbm.at[pl.ds(t0 + j * cc, cc), :], w_v[b], wsem[b])

        def wait_gather(b):
            pltpu.make_async_copy(t_hbm.at[si[b]], rows[b], gsem[b]).wait()
            if weighted:
                pltpu.make_async_copy(w_hbm.at[pl.ds(0, cc), :], w_v[b], wsem[b]).wait()

        def multiply(b):
            if not weighted:
                return

            def row(r, rc):
                wv = w_v[b][r, :]
                for u in range(nvec):
                    rows[b][r, pl.ds(u * 16, 16)] = rows[b][r, pl.ds(u * 16, 16)] * wv
                return rc
            lax.fori_loop(0, cc, row, 0)

        def slot(j, b, issue_next):
            wait_gather(b)
            multiply(b)
            if issue_next:
                load_idx(j + 1, 1 - b)
                issue_gather(j + 1, 1 - b)
            pltpu.sync_copy(rows[b], acc_sh.at[di[b]], add=True)

        load_idx(0, 0)
        issue_gather(0, 0)

        def group(g, carry):
            slot(2 * g, 0, True)
            slot(2 * g + 1, 1, True)
            return carry
        lax.fori_loop(0, kq - 1, group, 0)
        slot(k - 2, 0, True)
        slot(k - 1, 1, False)

        plsc.subcore_barrier()
        _writeout_rows(acc_sh, lambda r0, nr: out_hbm.at[c, pl.ds(r0, nr), :], s)

    scratch = [pltpu.VMEM((cc,), jnp.int32) for _ in range(4)]
    scratch += [pltpu.VMEM((cc, width), jnp.float32) for _ in range(2)]
    if weighted:
        scratch += [pltpu.VMEM((cc, 16), jnp.float32) for _ in range(2)]
    scratch.append(pltpu.VMEM_SHARED((_NACC, width), jnp.float32))
    scratch += [pltpu.SemaphoreType.DMA for _ in range(2)]
    if weighted:
        scratch += [pltpu.SemaphoreType.DMA for _ in range(2)]

    params = {}
    if width == 16:
        params["compiler_params"] = pltpu.CompilerParams(use_tc_tiling_on_sc=False)
    f = pl.kernel(
        body,
        out_type=jax.ShapeDtypeStruct((2, _N, width), jnp.float32),
        mesh=_mesh(),
        scratch_types=scratch,
        **params,
    )
    warg = w2d if weighted else jnp.zeros((1, 16), jnp.float32)
    return f(tbl, src_i, dst, warg)


def _sc_scatter16(vals16, dst):
    ep = dst.shape[0]
    k = ep // (_NC * _NS * _C)
    scratch = [
        pltpu.VMEM((_C,), jnp.int32),
        pltpu.VMEM((_C, 16), jnp.float32),
        pltpu.VMEM_SHARED((_NACC, 16), jnp.float32),
    ]

    def body(v_hbm, d_hbm, z_hbm, out_hbm, di_v, rows_v, acc_sh):
        c = lax.axis_index("c")
        s = lax.axis_index("s")
        zr = _NACC // _NS
        pltpu.sync_copy(z_hbm.at[pl.ds(s * zr, zr)], acc_sh.at[pl.ds(s * zr, zr)])
        plsc.subcore_barrier()

        def chunk(j, carry):
            off = (c * _NS + s) * (k * _C) + j * _C
            pltpu.sync_copy(d_hbm.at[pl.ds(off, _C)], di_v)
            pltpu.sync_copy(v_hbm.at[pl.ds(off, _C), :], rows_v)
            pltpu.sync_copy(rows_v, acc_sh.at[di_v], add=True)
            return carry
        lax.fori_loop(0, k, chunk, 0)
        plsc.subcore_barrier()
        _writeout_rows(acc_sh, lambda r0, nr: out_hbm.at[c, pl.ds(r0, nr), :], s)

    zeros = jnp.zeros((_NACC, 16), jnp.float32)
    f = pl.kernel(
        body,
        out_type=jax.ShapeDtypeStruct((2, _N, 16), jnp.float32),
        mesh=_mesh(),
        scratch_types=scratch,
    )
    return f(vals16, dst, zeros)


def _sc_gather_pair(ta, tb, ia_i, ib_i, width=128, edge_split=False):
    ep = ia_i.shape[-1]
    k = ep // (_NC * _NS * _C) if edge_split else ep // (_NS * _C)
    scratch = [
        pltpu.VMEM((_C,), jnp.int32),
        pltpu.VMEM((_C,), jnp.int32),
        pltpu.VMEM((_C, width), jnp.float32),
        pltpu.VMEM((_C, width), jnp.float32),
        pltpu.SemaphoreType.DMA,
        pltpu.SemaphoreType.DMA,
    ]

    def body(ta_hbm, tb_hbm, ia_hbm, ib_hbm, oa_hbm, ob_hbm,
             ia_v, ib_v, ra_v, rb_v, sema, semb):
        c = lax.axis_index("c")
        s = lax.axis_index("s")
        t0 = ((c * _NS + s) if edge_split else s) * (k * _C)

        def chunk(j, carry):
            off = t0 + j * _C
            if edge_split:
                pltpu.sync_copy(ia_hbm.at[pl.ds(off, _C)], ia_v)
                pltpu.sync_copy(ib_hbm.at[pl.ds(off, _C)], ib_v)
            else:
                pltpu.sync_copy(ia_hbm.at[c, pl.ds(off, _C)], ia_v)
                pltpu.sync_copy(ib_hbm.at[c, pl.ds(off, _C)], ib_v)
            da = pltpu.async_copy(ta_hbm.at[ia_v], ra_v, sema)
            db = pltpu.async_copy(tb_hbm.at[ib_v], rb_v, semb)
            da.wait()
            db.wait()
            if edge_split:
                pltpu.sync_copy(ra_v, oa_hbm.at[pl.ds(off, _C), :])
                pltpu.sync_copy(rb_v, ob_hbm.at[pl.ds(off, _C), :])
            else:
                pltpu.sync_copy(ra_v, oa_hbm.at[c, pl.ds(off, _C), :])
                pltpu.sync_copy(rb_v, ob_hbm.at[c, pl.ds(off, _C), :])
            return carry
        lax.fori_loop(0, k, chunk, 0)

    if edge_split:
        out_sd = jax.ShapeDtypeStruct((ep, width), jnp.float32)
    else:
        out_sd = jax.ShapeDtypeStruct((2, ep, width), jnp.float32)
    params = {}
    if width == 16:
        params["compiler_params"] = pltpu.CompilerParams(use_tc_tiling_on_sc=False)
    f = pl.kernel(
        body,
        out_type=(out_sd, out_sd),
        mesh=_mesh(),
        scratch_types=scratch,
        **params,
    )
    return f(ta, tb, ia_i, ib_i)


def _mm_h(xh, w, bias=None, relu=False, acc=None, row_scale=None):
    nb = _N // _MMBLK
    has_b = bias is not None
    has_a = acc is not None
    has_s = row_scale is not None

    def body(*refs):
        i = 0
        x_ref = refs[i]; i += 1
        w_ref = refs[i]; i += 1
        s_ref = refs[i] if has_s else None
        i += has_s
        b_ref = refs[i] if has_b else None
        i += has_b
        a_ref = refs[i] if has_a else None
        i += has_a
        o_ref = refs[i]
        ci = pl.program_id(2)
        xv = x_ref[...]
        if has_s:
            xv = xv * s_ref[...]
        contrib = jnp.dot(xv, w_ref[...], preferred_element_type=jnp.float32)

        @pl.when(ci == 0)
        def _():
            r = contrib
            if has_b:
                r = r + b_ref[...]
            if has_a:
                r = r + a_ref[...]
            o_ref[...] = r

        @pl.when(ci == 1)
        def _():
            r = o_ref[...] + contrib
            if relu:
                r = jnp.maximum(r, 0.0)
            o_ref[...] = r

    in_specs = [
        pl.BlockSpec((_MMBLK, 128), lambda i, co, ci: (ci * nb + i, 0)),
        pl.BlockSpec((128, 128), lambda i, co, ci: (ci, co)),
    ]
    args = [xh, w]
    if has_s:
        in_specs.append(pl.BlockSpec((_MMBLK, 1), lambda i, co, ci: (ci * nb + i, 0)))
        args.append(row_scale)
    if has_b:
        in_specs.append(pl.BlockSpec((1, 128), lambda i, co, ci: (0, co)))
        args.append(bias.reshape(1, 256))
    if has_a:
        in_specs.append(pl.BlockSpec((_MMBLK, 128), lambda i, co, ci: (co * nb + i, 0)))
        args.append(acc)
    return pl.pallas_call(
        body,
        grid=(nb, 2, 2),
        in_specs=in_specs,
        out_specs=pl.BlockSpec((_MMBLK, 128), lambda i, co, ci: (co * nb + i, 0)),
        out_shape=jax.ShapeDtypeStruct((2 * _N, 128), jnp.float32),
    )(*args)


def _mm_thin(xh, w16):
    nb = _N // _MMBLK

    def body(x0_ref, x1_ref, w_ref, o_ref):
        o_ref[...] = (
            jnp.dot(x0_ref[...], w_ref[0:128, :], preferred_element_type=jnp.float32)
            + jnp.dot(x1_ref[...], w_ref[128:256, :], preferred_element_type=jnp.float32))

    return pl.pallas_call(
        body,
        grid=(nb,),
        in_specs=[
            pl.BlockSpec((_MMBLK, 128), lambda i: (i, 0)),
            pl.BlockSpec((_MMBLK, 128), lambda i: (nb + i, 0)),
            pl.BlockSpec((256, 16), lambda i: (0, 0)),
        ],
        out_specs=pl.BlockSpec((_MMBLK, 16), lambda i: (i, 0)),
        out_shape=jax.ShapeDtypeStruct((_N, 16), jnp.float32),
    )(xh, xh, w16)


_EBLK = 1024


def _tc_score128(gl, gr, att):
    ep = gl.shape[1]
    ne = ep // _EBLK

    def body(gl0, gl1, gr0, gr1, att_ref, s_ref, m_ref):
        i = pl.program_id(0)
        t0 = gl0[0] + gr0[0]
        t1 = gl1[0] + gr1[0]
        t0 = jnp.where(t0 >= 0, t0, 0.2 * t0)
        t1 = jnp.where(t1 >= 0, t1, 0.2 * t1)
        s = (jnp.sum(t0 * att_ref[0:1, :], axis=-1, keepdims=True)
             + jnp.sum(t1 * att_ref[1:2, :], axis=-1, keepdims=True))
        s_ref[...] = s
        bm = jnp.max(s, keepdims=True)

        @pl.when(i == 0)
        def _():
            m_ref[...] = bm

        @pl.when(i > 0)
        def _():
            m_ref[...] = jnp.maximum(m_ref[...], bm)

    return pl.pallas_call(
        body,
        grid=(ne,),
        in_specs=[
            pl.BlockSpec((1, _EBLK, 128), lambda i: (0, i, 0)),
            pl.BlockSpec((1, _EBLK, 128), lambda i: (1, i, 0)),
            pl.BlockSpec((1, _EBLK, 128), lambda i: (0, i, 0)),
            pl.BlockSpec((1, _EBLK, 128), lambda i: (1, i, 0)),
            pl.BlockSpec((2, 128), lambda i: (0, 0)),
        ],
        out_specs=[
            pl.BlockSpec((_EBLK, 1), lambda i: (i, 0)),
            pl.BlockSpec((1, 1), lambda i: (0, 0)),
        ],
        out_shape=[
            jax.ShapeDtypeStruct((ep, 1), jnp.float32),
            jax.ShapeDtypeStruct((1, 1), jnp.float32),
        ],
    )(gl, gl, gr, gr, att.reshape(2, 128))


def _tc_score16(g1, g2, att0):
    ep = g1.shape[0]
    ne = ep // _EBLK

    def body(g1_ref, g2_ref, a_ref, s_ref, m_ref):
        i = pl.program_id(0)
        t = g1_ref[:, 0:1] + g2_ref[:, 1:2]
        t = jnp.where(t >= 0, t, 0.2 * t)
        s = t * a_ref[0, 0]
        s_ref[...] = s
        bm = jnp.max(s, keepdims=True)

        @pl.when(i == 0)
        def _():
            m_ref[...] = bm

        @pl.when(i > 0)
        def _():
            m_ref[...] = jnp.maximum(m_ref[...], bm)

    return pl.pallas_call(
        body,
        grid=(ne,),
        in_specs=[
            pl.BlockSpec((_EBLK, 16), lambda i: (i, 0)),
            pl.BlockSpec((_EBLK, 16), lambda i: (i, 0)),
            pl.BlockSpec((1, 1), lambda i: (0, 0)),
        ],
        out_specs=[
            pl.BlockSpec((_EBLK, 1), lambda i: (i, 0)),
            pl.BlockSpec((1, 1), lambda i: (0, 0)),
        ],
        out_shape=[
            jax.ShapeDtypeStruct((ep, 1), jnp.float32),
            jax.ShapeDtypeStruct((1, 1), jnp.float32),
        ],
    )(g1, g2, att0.reshape(1, 1))


def _tc_exp16(s, gmax):
    ep = s.shape[0]
    ne = ep // _EBLK

    def body(s_ref, m_ref, o_ref):
        e = jnp.exp(s_ref[...] - m_ref[0, 0])
        o_ref[...] = jnp.broadcast_to(e, (_EBLK, 16))

    return pl.pallas_call(
        body,
        grid=(ne,),
        in_specs=[
            pl.BlockSpec((_EBLK, 1), lambda i: (i, 0)),
            pl.BlockSpec((1, 1), lambda i: (0, 0)),
        ],
        out_specs=pl.BlockSpec((_EBLK, 16), lambda i: (i, 0)),
        out_shape=jax.ShapeDtypeStruct((ep, 16), jnp.float32),
    )(s, gmax)


def _tc_rowscale(a, s2, bias=None, relu=False, div=False):
    nb = _N // _MMBLK
    has_b = bias is not None

    def body(*refs):
        a_ref, s_ref = refs[0], refs[1]
        b_ref = refs[2] if has_b else None
        o_ref = refs[-1]
        v = a_ref[...] / s_ref[...] if div else a_ref[...] * s_ref[...]
        if has_b:
            v = v + b_ref[...]
        if relu:
            v = jnp.maximum(v, 0.0)
        o_ref[...] = v

    in_specs = [
        pl.BlockSpec((_MMBLK, 128), lambda i: (i, 0)),
        pl.BlockSpec((_MMBLK, 1), lambda i: (i, 0)),
    ]
    args = [a, s2]
    if has_b:
        in_specs.append(pl.BlockSpec((1, 128), lambda i: (0, i // nb)))
        args.append(bias.reshape(1, 256))
    return pl.pallas_call(
        body,
        grid=(2 * nb,),
        in_specs=in_specs,
        out_specs=pl.BlockSpec((_MMBLK, 128), lambda i: (i, 0)),
        out_shape=jax.ShapeDtypeStruct((2 * _N, 128), jnp.float32),
    )(*args)


def _tc_scale16(a, s):
    nb = _N // _MMBLK

    def body(a_ref, s_ref, o_ref):
        o_ref[...] = a_ref[...] * s_ref[...]

    return pl.pallas_call(
        body,
        grid=(nb,),
        in_specs=[
            pl.BlockSpec((_MMBLK, 16), lambda i: (i, 0)),
            pl.BlockSpec((_MMBLK, 1), lambda i: (i, 0)),
        ],
        out_specs=pl.BlockSpec((_MMBLK, 16), lambda i: (i, 0)),
        out_shape=jax.ShapeDtypeStruct((_N, 16), jnp.float32),
    )(a, s)


def _tc_scale16p(p, s):
    nb = _N // _MMBLK

    def body(p_ref, s_ref, o_ref):
        o_ref[...] = (p_ref[0] + p_ref[1]) * s_ref[...]

    return pl.pallas_call(
        body,
        grid=(nb,),
        in_specs=[
            pl.BlockSpec((2, _MMBLK, 16), lambda i: (0, i, 0)),
            pl.BlockSpec((_MMBLK, 1), lambda i: (i, 0)),
        ],
        out_specs=pl.BlockSpec((_MMBLK, 16), lambda i: (i, 0)),
        out_shape=jax.ShapeDtypeStruct((_N, 16), jnp.float32),
    )(p, s)


def _tc_prep(degp):
    nb = _N // _MMBLK

    def body(d_ref, dis_ref, ic_ref):
        deg = d_ref[0, :, 0:1] + d_ref[1, :, 0:1]
        dis = jnp.where(deg > 0, jax.lax.rsqrt(jnp.maximum(deg, 1e-12)), 0.0)
        dis_ref[...] = dis
        ic_ref[...] = 1.0 / jnp.maximum(deg, 1.0)

    return pl.pallas_call(
        body,
        grid=(nb,),
        in_specs=[pl.BlockSpec((2, _MMBLK, 16), lambda i: (0, i, 0))],
        out_specs=[
            pl.BlockSpec((_MMBLK, 1), lambda i: (i, 0)),
            pl.BlockSpec((_MMBLK, 1), lambda i: (i, 0)),
        ],
        out_shape=[
            jax.ShapeDtypeStruct((_N, 1), jnp.float32),
            jax.ShapeDtypeStruct((_N, 1), jnp.float32),
        ],
    )(degp)


def _tc_final(n4p, z4p, u_tag, p1, p2, p3, v_sage, hs_p, invcnt, scal, y, mask):
    nb = _N // _MMBLK

    def body(n4, z4, ut, p1r, p2r, p3r, vs, hs, ic, sc, y_ref, m_ref, o_ref):
        b4 = sc[0, 0]
        btag = sc[1, 0]
        bl3 = sc[2, 0]
        w0, w1, w2, blin = sc[3, 0], sc[4, 0], sc[5, 0], sc[6, 0]
        x1 = (n4[0, :, 0:1] + n4[1, :, 0:1]) / (z4[0, :, 0:1] + z4[1, :, 0:1]) + b4
        x2 = ut[:, 3:4] + p1r[:, 0:1] + p2r[:, 1:2] + p3r[:, 2:3] + btag
        x3 = (hs[0, :, 0:1] + hs[1, :, 0:1]) * ic[...] + bl3 + vs[:, 1:2]
        out = jnp.maximum(x1 * w0 + x2 * w1 + x3 * w2 + blin, 0.0)
        x_i = jnp.where(m_ref[...] != 0, out / 0.05, 0.0)
        o_ref[...] = jnp.where(y_ref[...] == 0.0, x_i, out)

    blk2 = pl.BlockSpec((2, _MMBLK, 16), lambda i: (0, i, 0))
    blk16 = pl.BlockSpec((_MMBLK, 16), lambda i: (i, 0))
    blk1 = pl.BlockSpec((_MMBLK, 1), lambda i: (i, 0))
    return pl.pallas_call(
        body,
        grid=(nb,),
        in_specs=[blk2, blk2, blk16, blk16, blk16, blk16, blk16, blk2,
                  blk1, pl.BlockSpec((7, 1), lambda i: (0, 0)), blk1, blk1],
        out_specs=blk1,
        out_shape=jax.ShapeDtypeStruct((_N, 1), jnp.float32),
    )(n4p, z4p, u_tag, p1, p2, p3, v_sage, hs_p, invcnt, scal, y, mask)


def _pad1(a, ep, fill=0):
    return jnp.concatenate(
        [a.astype(jnp.int32),
         jnp.full((ep - a.shape[0],), fill, jnp.int32)])


def _pad_trash(a, ep):
    npad = ep - a.shape[0]
    fill = _TRASH + (jnp.arange(npad, dtype=jnp.int32) % (_NACC - _N))
    return jnp.concatenate([a.astype(jnp.int32), fill])


def kernel(x, edge_index, y, params):
    n = _N
    src = edge_index[0].astype(jnp.int32)
    dst = edge_index[1].astype(jnp.int32)
    e = src.shape[0]
    loop = jnp.arange(n, dtype=jnp.int32)
    gran = _NC * _NS * _C * 4

    ep1 = ((e + gran - 1) // gran) * gran
    ep2 = ((e + n + gran - 1) // gran) * gran
    epr1, epr2 = ep1 // _C, ep2 // _C

    srcp1 = _pad1(src, ep1)
    dstp1 = _pad_trash(dst, ep1)
    src2_1 = jnp.stack([srcp1, srcp1 + n])

    src_sl = jnp.concatenate([src, loop])
    dst_sl = jnp.concatenate([dst, loop])
    srcp2 = _pad1(src_sl, ep2)
    dstp2 = _pad_trash(dst_sl, ep2)
    dstg2 = _pad1(dst_sl, ep2)
    src2_2 = jnp.stack([srcp2, srcp2 + n])
    dst2_2 = jnp.stack([dstg2, dstg2 + n])

    xh = jnp.transpose(x.reshape(n, 2, 128), (1, 0, 2)).reshape(2 * n, 128)

    ones16 = jnp.ones((ep1, 16), jnp.float32)
    degp = _sc_scatter16(ones16, dstp1)
    dis, invcnt = _tc_prep(degp)
    dis2 = jnp.concatenate([dis, dis], axis=0)
    dis2sq = dis2 * dis2
    invcnt2 = jnp.concatenate([invcnt, invcnt], axis=0)

    x1h = xh
    for p in params['gat'][:3]:
        xl = _mm_h(x1h, p['Wl'])
        xr = _mm_h(x1h, p['Wr'])
        gl, gr = _sc_gather_pair(xl, xr, src2_2, dst2_2, width=128)
        s, gmax = _tc_score128(gl, gr, p['att'])
        e16 = _tc_exp16(s, gmax)
        zp = _sc_scatter16(e16, dstp2)
        z = zp[0, :, 0:1] + zp[1, :, 0:1]
        z2 = jnp.concatenate([z, z], axis=0)
        numer = _sc_hop(xl, src2_2, dstp2, w2d=e16, width=128).reshape(2 * n, 128)
        x1h = _tc_rowscale(numer, z2, bias=p['b'], relu=True, div=True)

    p4 = params['gat'][3]
    w4 = jnp.concatenate(
        [p4['Wl'], p4['Wr'], jnp.zeros((256, 14), jnp.float32)], axis=1)
    t4 = _mm_thin(x1h, w4)
    g1, g2 = _sc_gather_pair(t4, t4, srcp2, dstg2, width=16, edge_split=True)
    s4, gmax4 = _tc_score16(g1, g2, p4['att'])
    e4 = _tc_exp16(s4, gmax4)
    z4p = _sc_scatter16(e4, dstp2)
    n4p = _sc_hop(t4, srcp2, dstp2, w2d=e4, width=16, edge_split=True)

    x2h = xh
    for li, p in enumerate(params['tag'][:2]):
        out = _mm_h(x2h, p['Ws'][0])
        hs = _tc_rowscale(x2h, dis2)
        for kk in range(1, 4):
            raw = _sc_hop(hs, src2_1, dstp1, width=128).reshape(2 * n, 128)
            last = kk == 3
            out = _mm_h(raw, p['Ws'][kk], row_scale=dis2, acc=out,
                        bias=p['b'] if last else None, relu=last)
            if not last:
                hs = _tc_rowscale(raw, dis2sq)
        x2h = out

    p3t = params['tag'][2]
    w16t = jnp.concatenate(
        [p3t['Ws'][1], p3t['Ws'][2], p3t['Ws'][3], p3t['Ws'][0],
         jnp.zeros((256, 12), jnp.float32)], axis=1)
    u_tag = _mm_thin(x2h, w16t)
    q = _tc_scale16(u_tag, dis)
    h1 = _sc_hop(q, srcp1, dstp1, width=16, edge_split=True)
    pp1 = _tc_scale16p(h1, dis)
    q = _tc_scale16(pp1, dis)
    h2 = _sc_hop(q, srcp1, dstp1, width=16, edge_split=True)
    pp2 = _tc_scale16p(h2, dis)
    q = _tc_scale16(pp2, dis)
    h3 = _sc_hop(q, srcp1, dstp1, width=16, edge_split=True)
    pp3 = _tc_scale16p(h3, dis)

    x3h = xh
    for p in params['sage'][:2]:
        raw = _sc_hop(x3h, src2_1, dstp1, width=128).reshape(2 * n, 128)
        out = _mm_h(raw, p['Wl'], row_scale=invcnt2, bias=p['bl'])
        x3h = _mm_h(x3h, p['Wr'], acc=out, relu=True)

    p3s = params['sage'][2]
    w16s = jnp.concatenate(
        [p3s['Wl'], p3s['Wr'], jnp.zeros((256, 14), jnp.float32)], axis=1)
    v_sage = _mm_thin(x3h, w16s)
    hs_p = _sc_hop(v_sage, srcp1, dstp1, width=16, edge_split=True)

    scal = jnp.stack([
        p4['b'][0], p3t['b'][0], p3s['bl'][0],
        params['lin']['W'][0, 0], params['lin']['W'][1, 0],
        params['lin']['W'][2, 0], params['lin']['b'][0],
    ]).reshape(7, 1)
    mask = jax.random.bernoulli(jax.random.key(42), 0.05, (n, 1)).astype(jnp.float32)
    return _tc_final(n4p, z4p, u_tag, pp1, pp2, pp3, v_sage, hs_p, invcnt,
                     scal, y, mask)


# speedup vs baseline: 1.2708x; 1.0445x over previous
import functools

import jax
import jax.numpy as jnp
from jax import lax
from jax.experimental import pallas as pl
from jax.experimental.pallas import tpu as pltpu
from jax.experimental.pallas import tpu_sc as plsc

_N = 10000
_NACC = 10240
_TRASH = 10000
_C = 128
_NS = 16
_NC = 2
_MMBLK = 1000


def _mesh():
    return plsc.VectorSubcoreMesh(core_axis_name="c", subcore_axis_name="s")


def _writeout_rows(acc_sh, out_slice_fn, s):
    @pl.when(s < _NS - 1)
    def _():
        pltpu.sync_copy(acc_sh.at[pl.ds(s * 640, 640)], out_slice_fn(s * 640, 640))

    @pl.when(s == _NS - 1)
    def _():
        pltpu.sync_copy(acc_sh.at[pl.ds(9600, 400)], out_slice_fn(9600, 400))


_D = 2


def _sc_hop(tbl, src_i, dst, w2d=None, width=128, edge_split=False):
    ep = dst.shape[0]
    k = ep // (_NC * _NS * _C) if edge_split else ep // (_NS * _C)
    weighted = w2d is not None
    nvec = width // 16

    def body(*refs):
        (t_hbm, s_hbm, d_hbm, w_hbm, out_hbm) = refs[:5]
        rest = list(refs[5:])
        si_v = rest.pop(0)
        di_v = rest.pop(0)
        rows_v = rest.pop(0)
        w_v = rest.pop(0) if weighted else None
        acc_sh = rest.pop(0)
        sem = rest.pop(0)
        wsem = rest.pop(0) if weighted else None

        c = lax.axis_index("c")
        s = lax.axis_index("s")
        t0 = ((c * _NS + s) if edge_split else s) * (k * _C)

        def zrow(r, rc):
            zv = jnp.zeros((16,), jnp.float32)
            for u in range(nvec):
                rows_v[r, pl.ds(u * 16, 16)] = zv
            return rc
        lax.fori_loop(0, _C, zrow, 0)
        zr = _NACC // _NS
        for zi in range(zr // _C):
            pltpu.sync_copy(rows_v, acc_sh.at[pl.ds(s * zr + zi * _C, _C)])
        plsc.subcore_barrier()

        def chunk(j, carry):
            off = t0 + j * _C
            if edge_split:
                pltpu.sync_copy(s_hbm.at[pl.ds(off, _C)], si_v)
            else:
                pltpu.sync_copy(s_hbm.at[c, pl.ds(off, _C)], si_v)
            pltpu.sync_copy(d_hbm.at[pl.ds(off, _C)], di_v)
            dg = pltpu.async_copy(t_hbm.at[si_v], rows_v, sem)
            if weighted:
                dw = pltpu.async_copy(w_hbm.at[pl.ds(off, _C), :], w_v, wsem)
            dg.wait()
            if weighted:
                dw.wait()

                def row(r, rc):
                    wv = w_v[r, :]
                    for u in range(nvec):
                        rows_v[r, pl.ds(u * 16, 16)] = rows_v[r, pl.ds(u * 16, 16)] * wv
                    return rc
                lax.fori_loop(0, _C, row, 0)
            pltpu.sync_copy(rows_v, acc_sh.at[di_v], add=True)
            return carry
        lax.fori_loop(0, k, chunk, 0)

        plsc.subcore_barrier()
        _writeout_rows(acc_sh, lambda r0, nr: out_hbm.at[c, pl.ds(r0, nr), :], s)

    scratch = [
        pltpu.VMEM((_C,), jnp.int32),
        pltpu.VMEM((_C,), jnp.int32),
        pltpu.VMEM((_C, width), jnp.float32),
    ]
    if weighted:
        scratch.append(pltpu.VMEM((_C, 16), jnp.float32))
    scratch.append(pltpu.VMEM_SHARED((_NACC, width), jnp.float32))
    scratch.append(pltpu.SemaphoreType.DMA)
    if weighted:
        scratch.append(pltpu.SemaphoreType.DMA)

    params = {}
    if width == 16:
        params["compiler_params"] = pltpu.CompilerParams(use_tc_tiling_on_sc=False)
    f = pl.kernel(
        body,
        out_type=jax.ShapeDtypeStruct((2, _N, width), jnp.float32),
        mesh=_mesh(),
        scratch_types=scratch,
        **params,
    )
    warg = w2d if weighted else jnp.zeros((1, 16), jnp.float32)
    return f(tbl, src_i, dst, warg)


def _sc_scatter16(vals16, dst):
    ep = dst.shape[0]
    k = ep // (_NC * _NS * _C)
    scratch = [
        pltpu.VMEM((_C,), jnp.int32),
        pltpu.VMEM((_C, 16), jnp.float32),
        pltpu.VMEM_SHARED((_NACC, 16), jnp.float32),
    ]

    def body(v_hbm, d_hbm, z_hbm, out_hbm, di_v, rows_v, acc_sh):
        c = lax.axis_index("c")
        s = lax.axis_index("s")
        zr = _NACC // _NS
        pltpu.sync_copy(z_hbm.at[pl.ds(s * zr, zr)], acc_sh.at[pl.ds(s * zr, zr)])
        plsc.subcore_barrier()

        def chunk(j, carry):
            off = (c * _NS + s) * (k * _C) + j * _C
            pltpu.sync_copy(d_hbm.at[pl.ds(off, _C)], di_v)
            pltpu.sync_copy(v_hbm.at[pl.ds(off, _C), :], rows_v)
            pltpu.sync_copy(rows_v, acc_sh.at[di_v], add=True)
            return carry
        lax.fori_loop(0, k, chunk, 0)
        plsc.subcore_barrier()
        _writeout_rows(acc_sh, lambda r0, nr: out_hbm.at[c, pl.ds(r0, nr), :], s)

    zeros = jnp.zeros((_NACC, 16), jnp.float32)
    f = pl.kernel(
        body,
        out_type=jax.ShapeDtypeStruct((2, _N, 16), jnp.float32),
        mesh=_mesh(),
        scratch_types=scratch,
    )
    return f(vals16, dst, zeros)


def _sc_gather_pair(ta, tb, ia_i, ib_i, width=128, edge_split=False):
    ep = ia_i.shape[-1]
    k = ep // (_NC * _NS * _C) if edge_split else ep // (_NS * _C)
    scratch = [
        pltpu.VMEM((_C,), jnp.int32),
        pltpu.VMEM((_C,), jnp.int32),
        pltpu.VMEM((_C, width), jnp.float32),
        pltpu.VMEM((_C, width), jnp.float32),
        pltpu.SemaphoreType.DMA,
        pltpu.SemaphoreType.DMA,
    ]

    def body(ta_hbm, tb_hbm, ia_hbm, ib_hbm, oa_hbm, ob_hbm,
             ia_v, ib_v, ra_v, rb_v, sema, semb):
        c = lax.axis_index("c")
        s = lax.axis_index("s")
        t0 = ((c * _NS + s) if edge_split else s) * (k * _C)

        def chunk(j, carry):
            off = t0 + j * _C
            if edge_split:
                pltpu.sync_copy(ia_hbm.at[pl.ds(off, _C)], ia_v)
                pltpu.sync_copy(ib_hbm.at[pl.ds(off, _C)], ib_v)
            else:
                pltpu.sync_copy(ia_hbm.at[c, pl.ds(off, _C)], ia_v)
                pltpu.sync_copy(ib_hbm.at[c, pl.ds(off, _C)], ib_v)
            da = pltpu.async_copy(ta_hbm.at[ia_v], ra_v, sema)
            db = pltpu.async_copy(tb_hbm.at[ib_v], rb_v, semb)
            da.wait()
            db.wait()
            if edge_split:
                pltpu.sync_copy(ra_v, oa_hbm.at[pl.ds(off, _C), :])
                pltpu.sync_copy(rb_v, ob_hbm.at[pl.ds(off, _C), :])
            else:
                pltpu.sync_copy(ra_v, oa_hbm.at[c, pl.ds(off, _C), :])
                pltpu.sync_copy(rb_v, ob_hbm.at[c, pl.ds(off, _C), :])
            return carry
        lax.fori_loop(0, k, chunk, 0)

    if edge_split:
        out_sd = jax.ShapeDtypeStruct((ep, width), jnp.float32)
    else:
        out_sd = jax.ShapeDtypeStruct((2, ep, width), jnp.float32)
    params = {}
    if width == 16:
        params["compiler_params"] = pltpu.CompilerParams(use_tc_tiling_on_sc=False)
    f = pl.kernel(
        body,
        out_type=(out_sd, out_sd),
        mesh=_mesh(),
        scratch_types=scratch,
        **params,
    )
    return f(ta, tb, ia_i, ib_i)


def _mm_h(xh, w, bias=None, relu=False, acc=None, row_scale=None):
    nb = _N // _MMBLK
    has_b = bias is not None
    has_a = acc is not None
    has_s = row_scale is not None

    def body(*refs):
        i = 0
        x_ref = refs[i]; i += 1
        w_ref = refs[i]; i += 1
        s_ref = refs[i] if has_s else None
        i += has_s
        b_ref = refs[i] if has_b else None
        i += has_b
        a_ref = refs[i] if has_a else None
        i += has_a
        o_ref = refs[i]
        ci = pl.program_id(2)
        xv = x_ref[...]
        if has_s:
            xv = xv * s_ref[...]
        contrib = jnp.dot(xv, w_ref[...], preferred_element_type=jnp.float32)

        @pl.when(ci == 0)
        def _():
            r = contrib
            if has_b:
                r = r + b_ref[...]
            if has_a:
                r = r + a_ref[...]
            o_ref[...] = r

        @pl.when(ci == 1)
        def _():
            r = o_ref[...] + contrib
            if relu:
                r = jnp.maximum(r, 0.0)
            o_ref[...] = r

    in_specs = [
        pl.BlockSpec((_MMBLK, 128), lambda i, co, ci: (ci * nb + i, 0)),
        pl.BlockSpec((128, 128), lambda i, co, ci: (ci, co)),
    ]
    args = [xh, w]
    if has_s:
        in_specs.append(pl.BlockSpec((_MMBLK, 1), lambda i, co, ci: (ci * nb + i, 0)))
        args.append(row_scale)
    if has_b:
        in_specs.append(pl.BlockSpec((1, 128), lambda i, co, ci: (0, co)))
        args.append(bias.reshape(1, 256))
    if has_a:
        in_specs.append(pl.BlockSpec((_MMBLK, 128), lambda i, co, ci: (co * nb + i, 0)))
        args.append(acc)
    return pl.pallas_call(
        body,
        grid=(nb, 2, 2),
        in_specs=in_specs,
        out_specs=pl.BlockSpec((_MMBLK, 128), lambda i, co, ci: (co * nb + i, 0)),
        out_shape=jax.ShapeDtypeStruct((2 * _N, 128), jnp.float32),
    )(*args)


def _mm_thin(xh, w16):
    nb = _N // _MMBLK

    def body(x0_ref, x1_ref, w_ref, o_ref):
        o_ref[...] = (
            jnp.dot(x0_ref[...], w_ref[0:128, :], preferred_element_type=jnp.float32)
            + jnp.dot(x1_ref[...], w_ref[128:256, :], preferred_element_type=jnp.float32))

    return pl.pallas_call(
        body,
        grid=(nb,),
        in_specs=[
            pl.BlockSpec((_MMBLK, 128), lambda i: (i, 0)),
            pl.BlockSpec((_MMBLK, 128), lambda i: (nb + i, 0)),
            pl.BlockSpec((256, 16), lambda i: (0, 0)),
        ],
        out_specs=pl.BlockSpec((_MMBLK, 16), lambda i: (i, 0)),
        out_shape=jax.ShapeDtypeStruct((_N, 16), jnp.float32),
    )(xh, xh, w16)


_EBLK = 1024


def _tc_score128(gl, gr, att):
    ep = gl.shape[1]
    ne = ep // _EBLK

    def body(gl0, gl1, gr0, gr1, att_ref, s_ref, m_ref):
        i = pl.program_id(0)
        t0 = gl0[0] + gr0[0]
        t1 = gl1[0] + gr1[0]
        t0 = jnp.where(t0 >= 0, t0, 0.2 * t0)
        t1 = jnp.where(t1 >= 0, t1, 0.2 * t1)
        s = (jnp.sum(t0 * att_ref[0:1, :], axis=-1, keepdims=True)
             + jnp.sum(t1 * att_ref[1:2, :], axis=-1, keepdims=True))
        s_ref[...] = s
        bm = jnp.max(s, keepdims=True)

        @pl.when(i == 0)
        def _():
            m_ref[...] = bm

        @pl.when(i > 0)
        def _():
            m_ref[...] = jnp.maximum(m_ref[...], bm)

    return pl.pallas_call(
        body,
        grid=(ne,),
        in_specs=[
            pl.BlockSpec((1, _EBLK, 128), lambda i: (0, i, 0)),
            pl.BlockSpec((1, _EBLK, 128), lambda i: (1, i, 0)),
            pl.BlockSpec((1, _EBLK, 128), lambda i: (0, i, 0)),
            pl.BlockSpec((1, _EBLK, 128), lambda i: (1, i, 0)),
            pl.BlockSpec((2, 128), lambda i: (0, 0)),
        ],
        out_specs=[
            pl.BlockSpec((_EBLK, 1), lambda i: (i, 0)),
            pl.BlockSpec((1, 1), lambda i: (0, 0)),
        ],
        out_shape=[
            jax.ShapeDtypeStruct((ep, 1), jnp.float32),
            jax.ShapeDtypeStruct((1, 1), jnp.float32),
        ],
    )(gl, gl, gr, gr, att.reshape(2, 128))


def _tc_score16(g1, g2, att0):
    ep = g1.shape[0]
    ne = ep // _EBLK

    def body(g1_ref, g2_ref, a_ref, s_ref, m_ref):
        i = pl.program_id(0)
        t = g1_ref[:, 0:1] + g2_ref[:, 1:2]
        t = jnp.where(t >= 0, t, 0.2 * t)
        s = t * a_ref[0, 0]
        s_ref[...] = s
        bm = jnp.max(s, keepdims=True)

        @pl.when(i == 0)
        def _():
            m_ref[...] = bm

        @pl.when(i > 0)
        def _():
            m_ref[...] = jnp.maximum(m_ref[...], bm)

    return pl.pallas_call(
        body,
        grid=(ne,),
        in_specs=[
            pl.BlockSpec((_EBLK, 16), lambda i: (i, 0)),
            pl.BlockSpec((_EBLK, 16), lambda i: (i, 0)),
            pl.BlockSpec((1, 1), lambda i: (0, 0)),
        ],
        out_specs=[
            pl.BlockSpec((_EBLK, 1), lambda i: (i, 0)),
            pl.BlockSpec((1, 1), lambda i: (0, 0)),
        ],
        out_shape=[
            jax.ShapeDtypeStruct((ep, 1), jnp.float32),
            jax.ShapeDtypeStruct((1, 1), jnp.float32),
        ],
    )(g1, g2, att0.reshape(1, 1))


def _tc_exp16(s, gmax):
    ep = s.shape[0]
    ne = ep // _EBLK

    def body(s_ref, m_ref, o_ref):
        e = jnp.exp(s_ref[...] - m_ref[0, 0])
        o_ref[...] = jnp.broadcast_to(e, (_EBLK, 16))

    return pl.pallas_call(
        body,
        grid=(ne,),
        in_specs=[
            pl.BlockSpec((_EBLK, 1), lambda i: (i, 0)),
            pl.BlockSpec((1, 1), lambda i: (0, 0)),
        ],
        out_specs=pl.BlockSpec((_EBLK, 16), lambda i: (i, 0)),
        out_shape=jax.ShapeDtypeStruct((ep, 16), jnp.float32),
    )(s, gmax)


def _tc_rowscale(a, s2, bias=None, relu=False, div=False):
    nb = _N // _MMBLK
    has_b = bias is not None

    def body(*refs):
        a_ref, s_ref = refs[0], refs[1]
        b_ref = refs[2] if has_b else None
        o_ref = refs[-1]
        v = a_ref[...] / s_ref[...] if div else a_ref[...] * s_ref[...]
        if has_b:
            v = v + b_ref[...]
        if relu:
            v = jnp.maximum(v, 0.0)
        o_ref[...] = v

    in_specs = [
        pl.BlockSpec((_MMBLK, 128), lambda i: (i, 0)),
        pl.BlockSpec((_MMBLK, 1), lambda i: (i, 0)),
    ]
    args = [a, s2]
    if has_b:
        in_specs.append(pl.BlockSpec((1, 128), lambda i: (0, i // nb)))
        args.append(bias.reshape(1, 256))
    return pl.pallas_call(
        body,
        grid=(2 * nb,),
        in_specs=in_specs,
        out_specs=pl.BlockSpec((_MMBLK, 128), lambda i: (i, 0)),
        out_shape=jax.ShapeDtypeStruct((2 * _N, 128), jnp.float32),
    )(*args)


def _tc_scale16(a, s):
    nb = _N // _MMBLK

    def body(a_ref, s_ref, o_ref):
        o_ref[...] = a_ref[...] * s_ref[...]

    return pl.pallas_call(
        body,
        grid=(nb,),
        in_specs=[
            pl.BlockSpec((_MMBLK, 16), lambda i: (i, 0)),
            pl.BlockSpec((_MMBLK, 1), lambda i: (i, 0)),
        ],
        out_specs=pl.BlockSpec((_MMBLK, 16), lambda i: (i, 0)),
        out_shape=jax.ShapeDtypeStruct((_N, 16), jnp.float32),
    )(a, s)


def _tc_scale16p(p, s):
    nb = _N // _MMBLK

    def body(p_ref, s_ref, o_ref):
        o_ref[...] = (p_ref[0] + p_ref[1]) * s_ref[...]

    return pl.pallas_call(
        body,
        grid=(nb,),
        in_specs=[
            pl.BlockSpec((2, _MMBLK, 16), lambda i: (0, i, 0)),
            pl.BlockSpec((_MMBLK, 1), lambda i: (i, 0)),
        ],
        out_specs=pl.BlockSpec((_MMBLK, 16), lambda i: (i, 0)),
        out_shape=jax.ShapeDtypeStruct((_N, 16), jnp.float32),
    )(p, s)


def _tc_prep(degp):
    nb = _N // _MMBLK

    def body(d_ref, dis_ref, ic_ref):
        deg = d_ref[0, :, 0:1] + d_ref[1, :, 0:1]
        dis = jnp.where(deg > 0, jax.lax.rsqrt(jnp.maximum(deg, 1e-12)), 0.0)
        dis_ref[...] = dis
        ic_ref[...] = 1.0 / jnp.maximum(deg, 1.0)

    return pl.pallas_call(
        body,
        grid=(nb,),
        in_specs=[pl.BlockSpec((2, _MMBLK, 16), lambda i: (0, i, 0))],
        out_specs=[
            pl.BlockSpec((_MMBLK, 1), lambda i: (i, 0)),
            pl.BlockSpec((_MMBLK, 1), lambda i: (i, 0)),
        ],
        out_shape=[
            jax.ShapeDtypeStruct((_N, 1), jnp.float32),
            jax.ShapeDtypeStruct((_N, 1), jnp.float32),
        ],
    )(degp)


def _tc_final(n4p, z4p, u_tag, p1, p2, p3, v_sage, hs_p, invcnt, scal, y, mask):
    nb = _N // _MMBLK

    def body(n4, z4, ut, p1r, p2r, p3r, vs, hs, ic, sc, y_ref, m_ref, o_ref):
        b4 = sc[0, 0]
        btag = sc[1, 0]
        bl3 = sc[2, 0]
        w0, w1, w2, blin = sc[3, 0], sc[4, 0], sc[5, 0], sc[6, 0]
        x1 = (n4[0, :, 0:1] + n4[1, :, 0:1]) / (z4[0, :, 0:1] + z4[1, :, 0:1]) + b4
        x2 = ut[:, 3:4] + p1r[:, 0:1] + p2r[:, 1:2] + p3r[:, 2:3] + btag
        x3 = (hs[0, :, 0:1] + hs[1, :, 0:1]) * ic[...] + bl3 + vs[:, 1:2]
        out = jnp.maximum(x1 * w0 + x2 * w1 + x3 * w2 + blin, 0.0)
        x_i = jnp.where(m_ref[...] != 0, out / 0.05, 0.0)
        o_ref[...] = jnp.where(y_ref[...] == 0.0, x_i, out)

    blk2 = pl.BlockSpec((2, _MMBLK, 16), lambda i: (0, i, 0))
    blk16 = pl.BlockSpec((_MMBLK, 16), lambda i: (i, 0))
    blk1 = pl.BlockSpec((_MMBLK, 1), lambda i: (i, 0))
    return pl.pallas_call(
        body,
        grid=(nb,),
        in_specs=[blk2, blk2, blk16, blk16, blk16, blk16, blk16, blk2,
                  blk1, pl.BlockSpec((7, 1), lambda i: (0, 0)), blk1, blk1],
        out_specs=blk1,
        out_shape=jax.ShapeDtypeStruct((_N, 1), jnp.float32),
    )(n4p, z4p, u_tag, p1, p2, p3, v_sage, hs_p, invcnt, scal, y, mask)


def _pad1(a, ep, fill=0):
    return jnp.concatenate(
        [a.astype(jnp.int32),
         jnp.full((ep - a.shape[0],), fill, jnp.int32)])


def _pad_trash(a, ep):
    npad = ep - a.shape[0]
    fill = _TRASH + (jnp.arange(npad, dtype=jnp.int32) % (_NACC - _N))
    return jnp.concatenate([a.astype(jnp.int32), fill])


def kernel(x, edge_index, y, params):
    n = _N
    src = edge_index[0].astype(jnp.int32)
    dst = edge_index[1].astype(jnp.int32)
    e = src.shape[0]
    loop = jnp.arange(n, dtype=jnp.int32)
    gran = _NC * _NS * _C * 4

    ep1 = ((e + gran - 1) // gran) * gran
    ep2 = ((e + n + gran - 1) // gran) * gran
    epr1, epr2 = ep1 // _C, ep2 // _C

    srcp1 = _pad1(src, ep1)
    dstp1 = _pad_trash(dst, ep1)
    src2_1 = jnp.stack([srcp1, srcp1 + n])

    src_sl = jnp.concatenate([src, loop])
    dst_sl = jnp.concatenate([dst, loop])
    srcp2 = _pad1(src_sl, ep2)
    dstp2 = _pad_trash(dst_sl, ep2)
    dstg2 = _pad1(dst_sl, ep2)
    src2_2 = jnp.stack([srcp2, srcp2 + n])
    dst2_2 = jnp.stack([dstg2, dstg2 + n])

    xh = jnp.transpose(x.reshape(n, 2, 128), (1, 0, 2)).reshape(2 * n, 128)

    ones16 = jnp.ones((ep1, 16), jnp.float32)
    degp = _sc_scatter16(ones16, dstp1)
    dis, invcnt = _tc_prep(degp)
    dis2 = jnp.concatenate([dis, dis], axis=0)
    dis2sq = dis2 * dis2
    invcnt2 = jnp.concatenate([invcnt, invcnt], axis=0)

    x1h = xh
    for p in params['gat'][:3]:
        xl = _mm_h(x1h, p['Wl'])
        xr = _mm_h(x1h, p['Wr'])
        gl, gr = _sc_gather_pair(xl, xr, src2_2, dst2_2, width=128)
        s, gmax = _tc_score128(gl, gr, p['att'])
        e16 = _tc_exp16(s, gmax)
        zp = _sc_scatter16(e16, dstp2)
        z = zp[0, :, 0:1] + zp[1, :, 0:1]
        z2 = jnp.concatenate([z, z], axis=0)
        numer = _sc_hop(xl, src2_2, dstp2, w2d=e16, width=128).reshape(2 * n, 128)
        x1h = _tc_rowscale(numer, z2, bias=p['b'], relu=True, div=True)

    p4 = params['gat'][3]
    w4 = jnp.concatenate(
        [p4['Wl'], p4['Wr'], jnp.zeros((256, 14), jnp.float32)], axis=1)
    t4 = _mm_thin(x1h, w4)
    g1, g2 = _sc_gather_pair(t4, t4, srcp2, dstg2, width=16, edge_split=True)
    s4, gmax4 = _tc_score16(g1, g2, p4['att'])
    e4 = _tc_exp16(s4, gmax4)
    z4p = _sc_scatter16(e4, dstp2)
    n4p = _sc_hop(t4, srcp2, dstp2, w2d=e4, width=16, edge_split=True)

    x2h = xh
    for li, p in enumerate(params['tag'][:2]):
        out = _mm_h(x2h, p['Ws'][0])
        hs = _tc_rowscale(x2h, dis2)
        for kk in range(1, 4):
            raw = _sc_hop(hs, src2_1, dstp1, width=128).reshape(2 * n, 128)
            last = kk == 3
            out = _mm_h(raw, p['Ws'][kk], row_scale=dis2, acc=out,
                        bias=p['b'] if last else None, relu=last)
            if not last:
                hs = _tc_rowscale(raw, dis2sq)
        x2h = out

    p3t = params['tag'][2]
    w16t = jnp.concatenate(
        [p3t['Ws'][1], p3t['Ws'][2], p3t['Ws'][3], p3t['Ws'][0],
         jnp.zeros((256, 12), jnp.float32)], axis=1)
    u_tag = _mm_thin(x2h, w16t)
    q = _tc_scale16(u_tag, dis)
    h1 = _sc_hop(q, srcp1, dstp1, width=16, edge_split=True)
    pp1 = _tc_scale16p(h1, dis)
    q = _tc_scale16(pp1, dis)
    h2 = _sc_hop(q, srcp1, dstp1, width=16, edge_split=True)
    pp2 = _tc_scale16p(h2, dis)
    q = _tc_scale16(pp2, dis)
    h3 = _sc_hop(q, srcp1, dstp1, width=16, edge_split=True)
    pp3 = _tc_scale16p(h3, dis)

    x3h = xh
    for p in params['sage'][:2]:
        raw = _sc_hop(x3h, src2_1, dstp1, width=128).reshape(2 * n, 128)
        out = _mm_h(raw, p['Wl'], row_scale=invcnt2, bias=p['bl'])
        x3h = _mm_h(x3h, p['Wr'], acc=out, relu=True)

    p3s = params['sage'][2]
    w16s = jnp.concatenate(
        [p3s['Wl'], p3s['Wr'], jnp.zeros((256, 14), jnp.float32)], axis=1)
    v_sage = _mm_thin(x3h, w16s)
    hs_p = _sc_hop(v_sage, srcp1, dstp1, width=16, edge_split=True)

    scal = jnp.stack([
        p4['b'][0], p3t['b'][0], p3s['bl'][0],
        params['lin']['W'][0, 0], params['lin']['W'][1, 0],
        params['lin']['W'][2, 0], params['lin']['b'][0],
    ]).reshape(7, 1)
    mask = jax.random.bernoulli(jax.random.key(42), 0.05, (n, 1)).astype(jnp.float32)
    return _tc_final(n4p, z4p, u_tag, pp1, pp2, pp3, v_sage, hs_p, invcnt,
                     scal, y, mask)


# speedup vs baseline: 1.7702x; 1.3929x over previous
import functools

import jax
import jax.numpy as jnp
from jax import lax
from jax.experimental import pallas as pl
from jax.experimental.pallas import tpu as pltpu
from jax.experimental.pallas import tpu_sc as plsc

_N = 10000
_NACC = 10240
_TRASH = 10000
_C = 128
_NS = 16
_NC = 2
_MMBLK = 1000


def _mesh():
    return plsc.VectorSubcoreMesh(core_axis_name="c", subcore_axis_name="s")


def _writeout_rows(acc_sh, out_slice_fn, s):
    @pl.when(s < _NS - 1)
    def _():
        pltpu.sync_copy(acc_sh.at[pl.ds(s * 640, 640)], out_slice_fn(s * 640, 640))

    @pl.when(s == _NS - 1)
    def _():
        pltpu.sync_copy(acc_sh.at[pl.ds(9600, 400)], out_slice_fn(9600, 400))


_D = 2


def _sc_hop(tbl, src_i, dst, w2d=None, width=128, edge_split=False):
    ep = dst.shape[0]
    k = ep // (_NC * _NS * _C) if edge_split else ep // (_NS * _C)
    weighted = w2d is not None
    nvec = width // 16

    def body(*refs):
        (t_hbm, s_hbm, d_hbm, w_hbm, z_hbm, out_hbm) = refs[:6]
        rest = list(refs[6:])
        si_v = rest.pop(0)
        di_v = rest.pop(0)
        rows_v = rest.pop(0)
        w_v = rest.pop(0) if weighted else None
        acc_sh = rest.pop(0)
        sem = rest.pop(0)

        c = lax.axis_index("c")
        s = lax.axis_index("s")
        t0 = ((c * _NS + s) if edge_split else s) * (k * _C)
        zr = _NACC // _NS
        pltpu.sync_copy(z_hbm.at[pl.ds(s * zr, zr)], acc_sh.at[pl.ds(s * zr, zr)])
        plsc.subcore_barrier()

        def chunk(j, carry):
            off = t0 + j * _C
            if edge_split:
                pltpu.sync_copy(s_hbm.at[pl.ds(off, _C)], si_v)
            else:
                pltpu.sync_copy(s_hbm.at[c, pl.ds(off, _C)], si_v)
            pltpu.sync_copy(d_hbm.at[pl.ds(off, _C)], di_v)
            pltpu.async_copy(t_hbm.at[si_v], rows_v, sem).wait()
            if weighted:
                pltpu.sync_copy(w_hbm.at[pl.ds(off, _C), :], w_v)

                def row(r, rc):
                    wv = w_v[r, :]
                    for u in range(nvec):
                        rows_v[r, pl.ds(u * 16, 16)] = rows_v[r, pl.ds(u * 16, 16)] * wv
                    return rc
                lax.fori_loop(0, _C, row, 0)
            pltpu.sync_copy(rows_v, acc_sh.at[di_v], add=True)
            return carry
        lax.fori_loop(0, k, chunk, 0)

        plsc.subcore_barrier()
        _writeout_rows(acc_sh, lambda r0, nr: out_hbm.at[c, pl.ds(r0, nr), :], s)

    scratch = [
        pltpu.VMEM((_C,), jnp.int32),
        pltpu.VMEM((_C,), jnp.int32),
        pltpu.VMEM((_C, width), jnp.float32),
    ]
    if weighted:
        scratch.append(pltpu.VMEM((_C, 16), jnp.float32))
    scratch.append(pltpu.VMEM_SHARED((_NACC, width), jnp.float32))
    scratch.append(pltpu.SemaphoreType.DMA)

    params = {}
    if width == 16:
        params["compiler_params"] = pltpu.CompilerParams(use_tc_tiling_on_sc=False)
    f = pl.kernel(
        body,
        out_type=jax.ShapeDtypeStruct((2, _N, width), jnp.float32),
        mesh=_mesh(),
        scratch_types=scratch,
        **params,
    )
    warg = w2d if weighted else jnp.zeros((1, 16), jnp.float32)
    zeros = jnp.zeros((_NACC, width), jnp.float32)
    return f(tbl, src_i, dst, warg, zeros)


def _sc_scatter16(vals16, dst):
    ep = dst.shape[0]
    k = ep // (_NC * _NS * _C)
    scratch = [
        pltpu.VMEM((_C,), jnp.int32),
        pltpu.VMEM((_C, 16), jnp.float32),
        pltpu.VMEM_SHARED((_NACC, 16), jnp.float32),
    ]

    def body(v_hbm, d_hbm, z_hbm, out_hbm, di_v, rows_v, acc_sh):
        c = lax.axis_index("c")
        s = lax.axis_index("s")
        zr = _NACC // _NS
        pltpu.sync_copy(z_hbm.at[pl.ds(s * zr, zr)], acc_sh.at[pl.ds(s * zr, zr)])
        plsc.subcore_barrier()

        def chunk(j, carry):
            off = (c * _NS + s) * (k * _C) + j * _C
            pltpu.sync_copy(d_hbm.at[pl.ds(off, _C)], di_v)
            pltpu.sync_copy(v_hbm.at[pl.ds(off, _C), :], rows_v)
            pltpu.sync_copy(rows_v, acc_sh.at[di_v], add=True)
            return carry
        lax.fori_loop(0, k, chunk, 0)
        plsc.subcore_barrier()
        _writeout_rows(acc_sh, lambda r0, nr: out_hbm.at[c, pl.ds(r0, nr), :], s)

    zeros = jnp.zeros((_NACC, 16), jnp.float32)
    f = pl.kernel(
        body,
        out_type=jax.ShapeDtypeStruct((2, _N, 16), jnp.float32),
        mesh=_mesh(),
        scratch_types=scratch,
    )
    return f(vals16, dst, zeros)


def _sc_gather_pair(ta, tb, ia_i, ib_i, width=128, edge_split=False):
    ep = ia_i.shape[-1]
    k = ep // (_NC * _NS * _C) if edge_split else ep // (_NS * _C)
    scratch = [
        pltpu.VMEM((_C,), jnp.int32),
        pltpu.VMEM((_C,), jnp.int32),
        pltpu.VMEM((_C, width), jnp.float32),
        pltpu.VMEM((_C, width), jnp.float32),
        pltpu.SemaphoreType.DMA,
        pltpu.SemaphoreType.DMA,
    ]

    def body(ta_hbm, tb_hbm, ia_hbm, ib_hbm, oa_hbm, ob_hbm,
             ia_v, ib_v, ra_v, rb_v, sema, semb):
        c = lax.axis_index("c")
        s = lax.axis_index("s")
        t0 = ((c * _NS + s) if edge_split else s) * (k * _C)

        def chunk(j, carry):
            off = t0 + j * _C
            if edge_split:
                pltpu.sync_copy(ia_hbm.at[pl.ds(off, _C)], ia_v)
                pltpu.sync_copy(ib_hbm.at[pl.ds(off, _C)], ib_v)
            else:
                pltpu.sync_copy(ia_hbm.at[c, pl.ds(off, _C)], ia_v)
                pltpu.sync_copy(ib_hbm.at[c, pl.ds(off, _C)], ib_v)
            da = pltpu.async_copy(ta_hbm.at[ia_v], ra_v, sema)
            db = pltpu.async_copy(tb_hbm.at[ib_v], rb_v, semb)
            da.wait()
            db.wait()
            if edge_split:
                pltpu.sync_copy(ra_v, oa_hbm.at[pl.ds(off, _C), :])
                pltpu.sync_copy(rb_v, ob_hbm.at[pl.ds(off, _C), :])
            else:
                pltpu.sync_copy(ra_v, oa_hbm.at[c, pl.ds(off, _C), :])
                pltpu.sync_copy(rb_v, ob_hbm.at[c, pl.ds(off, _C), :])
            return carry
        lax.fori_loop(0, k, chunk, 0)

    if edge_split:
        out_sd = jax.ShapeDtypeStruct((ep, width), jnp.float32)
    else:
        out_sd = jax.ShapeDtypeStruct((2, ep, width), jnp.float32)
    params = {}
    if width == 16:
        params["compiler_params"] = pltpu.CompilerParams(use_tc_tiling_on_sc=False)
    f = pl.kernel(
        body,
        out_type=(out_sd, out_sd),
        mesh=_mesh(),
        scratch_types=scratch,
        **params,
    )
    return f(ta, tb, ia_i, ib_i)


def _mm_h(xh, w, bias=None, relu=False, acc=None, row_scale=None):
    nb = _N // _MMBLK
    has_b = bias is not None
    has_a = acc is not None
    has_s = row_scale is not None

    def body(*refs):
        i = 0
        x_ref = refs[i]; i += 1
        w_ref = refs[i]; i += 1
        s_ref = refs[i] if has_s else None
        i += has_s
        b_ref = refs[i] if has_b else None
        i += has_b
        a_ref = refs[i] if has_a else None
        i += has_a
        o_ref = refs[i]
        ci = pl.program_id(2)
        xv = x_ref[...]
        if has_s:
            xv = xv * s_ref[...]
        contrib = jnp.dot(xv, w_ref[...], preferred_element_type=jnp.float32)

        @pl.when(ci == 0)
        def _():
            r = contrib
            if has_b:
                r = r + b_ref[...]
            if has_a:
                r = r + a_ref[...]
            o_ref[...] = r

        @pl.when(ci == 1)
        def _():
            r = o_ref[...] + contrib
            if relu:
                r = jnp.maximum(r, 0.0)
            o_ref[...] = r

    in_specs = [
        pl.BlockSpec((_MMBLK, 128), lambda i, co, ci: (ci * nb + i, 0)),
        pl.BlockSpec((128, 128), lambda i, co, ci: (ci, co)),
    ]
    args = [xh, w]
    if has_s:
        in_specs.append(pl.BlockSpec((_MMBLK, 1), lambda i, co, ci: (ci * nb + i, 0)))
        args.append(row_scale)
    if has_b:
        in_specs.append(pl.BlockSpec((1, 128), lambda i, co, ci: (0, co)))
        args.append(bias.reshape(1, 256))
    if has_a:
        in_specs.append(pl.BlockSpec((_MMBLK, 128), lambda i, co, ci: (co * nb + i, 0)))
        args.append(acc)
    return pl.pallas_call(
        body,
        grid=(nb, 2, 2),
        in_specs=in_specs,
        out_specs=pl.BlockSpec((_MMBLK, 128), lambda i, co, ci: (co * nb + i, 0)),
        out_shape=jax.ShapeDtypeStruct((2 * _N, 128), jnp.float32),
    )(*args)


def _mm_thin(xh, w16):
    nb = _N // _MMBLK

    def body(x0_ref, x1_ref, w_ref, o_ref):
        o_ref[...] = (
            jnp.dot(x0_ref[...], w_ref[0:128, :], preferred_element_type=jnp.float32)
            + jnp.dot(x1_ref[...], w_ref[128:256, :], preferred_element_type=jnp.float32))

    return pl.pallas_call(
        body,
        grid=(nb,),
        in_specs=[
            pl.BlockSpec((_MMBLK, 128), lambda i: (i, 0)),
            pl.BlockSpec((_MMBLK, 128), lambda i: (nb + i, 0)),
            pl.BlockSpec((256, 16), lambda i: (0, 0)),
        ],
        out_specs=pl.BlockSpec((_MMBLK, 16), lambda i: (i, 0)),
        out_shape=jax.ShapeDtypeStruct((_N, 16), jnp.float32),
    )(xh, xh, w16)


_EBLK = 1024


def _tc_score128(gl, gr, att):
    ep = gl.shape[1]
    ne = ep // _EBLK

    def body(gl0, gl1, gr0, gr1, att_ref, s_ref, m_ref):
        i = pl.program_id(0)
        t0 = gl0[0] + gr0[0]
        t1 = gl1[0] + gr1[0]
        t0 = jnp.where(t0 >= 0, t0, 0.2 * t0)
        t1 = jnp.where(t1 >= 0, t1, 0.2 * t1)
        s = (jnp.sum(t0 * att_ref[0:1, :], axis=-1, keepdims=True)
             + jnp.sum(t1 * att_ref[1:2, :], axis=-1, keepdims=True))
        s_ref[...] = s
        bm = jnp.max(s, keepdims=True)

        @pl.when(i == 0)
        def _():
            m_ref[...] = bm

        @pl.when(i > 0)
        def _():
            m_ref[...] = jnp.maximum(m_ref[...], bm)

    return pl.pallas_call(
        body,
        grid=(ne,),
        in_specs=[
            pl.BlockSpec((1, _EBLK, 128), lambda i: (0, i, 0)),
            pl.BlockSpec((1, _EBLK, 128), lambda i: (1, i, 0)),
            pl.BlockSpec((1, _EBLK, 128), lambda i: (0, i, 0)),
            pl.BlockSpec((1, _EBLK, 128), lambda i: (1, i, 0)),
            pl.BlockSpec((2, 128), lambda i: (0, 0)),
        ],
        out_specs=[
            pl.BlockSpec((_EBLK, 1), lambda i: (i, 0)),
            pl.BlockSpec((1, 1), lambda i: (0, 0)),
        ],
        out_shape=[
            jax.ShapeDtypeStruct((ep, 1), jnp.float32),
            jax.ShapeDtypeStruct((1, 1), jnp.float32),
        ],
    )(gl, gl, gr, gr, att.reshape(2, 128))


def _tc_score16(g1, g2, att0):
    ep = g1.shape[0]
    ne = ep // _EBLK

    def body(g1_ref, g2_ref, a_ref, s_ref, m_ref):
        i = pl.program_id(0)
        t = g1_ref[:, 0:1] + g2_ref[:, 1:2]
        t = jnp.where(t >= 0, t, 0.2 * t)
        s = t * a_ref[0, 0]
        s_ref[...] = s
        bm = jnp.max(s, keepdims=True)

        @pl.when(i == 0)
        def _():
            m_ref[...] = bm

        @pl.when(i > 0)
        def _():
            m_ref[...] = jnp.maximum(m_ref[...], bm)

    return pl.pallas_call(
        body,
        grid=(ne,),
        in_specs=[
            pl.BlockSpec((_EBLK, 16), lambda i: (i, 0)),
            pl.BlockSpec((_EBLK, 16), lambda i: (i, 0)),
            pl.BlockSpec((1, 1), lambda i: (0, 0)),
        ],
        out_specs=[
            pl.BlockSpec((_EBLK, 1), lambda i: (i, 0)),
            pl.BlockSpec((1, 1), lambda i: (0, 0)),
        ],
        out_shape=[
            jax.ShapeDtypeStruct((ep, 1), jnp.float32),
            jax.ShapeDtypeStruct((1, 1), jnp.float32),
        ],
    )(g1, g2, att0.reshape(1, 1))


def _tc_exp16(s, gmax):
    ep = s.shape[0]
    ne = ep // _EBLK

    def body(s_ref, m_ref, o_ref):
        e = jnp.exp(s_ref[...] - m_ref[0, 0])
        o_ref[...] = jnp.broadcast_to(e, (_EBLK, 16))

    return pl.pallas_call(
        body,
        grid=(ne,),
        in_specs=[
            pl.BlockSpec((_EBLK, 1), lambda i: (i, 0)),
            pl.BlockSpec((1, 1), lambda i: (0, 0)),
        ],
        out_specs=pl.BlockSpec((_EBLK, 16), lambda i: (i, 0)),
        out_shape=jax.ShapeDtypeStruct((ep, 16), jnp.float32),
    )(s, gmax)


def _tc_rowscale(a, s2, bias=None, relu=False, div=False):
    nb = _N // _MMBLK
    has_b = bias is not None

    def body(*refs):
        a_ref, s_ref = refs[0], refs[1]
        b_ref = refs[2] if has_b else None
        o_ref = refs[-1]
        v = a_ref[...] / s_ref[...] if div else a_ref[...] * s_ref[...]
        if has_b:
            v = v + b_ref[...]
        if relu:
            v = jnp.maximum(v, 0.0)
        o_ref[...] = v

    in_specs = [
        pl.BlockSpec((_MMBLK, 128), lambda i: (i, 0)),
        pl.BlockSpec((_MMBLK, 1), lambda i: (i, 0)),
    ]
    args = [a, s2]
    if has_b:
        in_specs.append(pl.BlockSpec((1, 128), lambda i: (0, i // nb)))
        args.append(bias.reshape(1, 256))
    return pl.pallas_call(
        body,
        grid=(2 * nb,),
        in_specs=in_specs,
        out_specs=pl.BlockSpec((_MMBLK, 128), lambda i: (i, 0)),
        out_shape=jax.ShapeDtypeStruct((2 * _N, 128), jnp.float32),
    )(*args)


def _tc_scale16(a, s):
    nb = _N // _MMBLK

    def body(a_ref, s_ref, o_ref):
        o_ref[...] = a_ref[...] * s_ref[...]

    return pl.pallas_call(
        body,
        grid=(nb,),
        in_specs=[
            pl.BlockSpec((_MMBLK, 16), lambda i: (i, 0)),
            pl.BlockSpec((_MMBLK, 1), lambda i: (i, 0)),
        ],
        out_specs=pl.BlockSpec((_MMBLK, 16), lambda i: (i, 0)),
        out_shape=jax.ShapeDtypeStruct((_N, 16), jnp.float32),
    )(a, s)


def _tc_scale16p(p, s):
    nb = _N // _MMBLK

    def body(p_ref, s_ref, o_ref):
        o_ref[...] = (p_ref[0] + p_ref[1]) * s_ref[...]

    return pl.pallas_call(
        body,
        grid=(nb,),
        in_specs=[
            pl.BlockSpec((2, _MMBLK, 16), lambda i: (0, i, 0)),
            pl.BlockSpec((_MMBLK, 1), lambda i: (i, 0)),
        ],
        out_specs=pl.BlockSpec((_MMBLK, 16), lambda i: (i, 0)),
        out_shape=jax.ShapeDtypeStruct((_N, 16), jnp.float32),
    )(p, s)


def _tc_prep(degp):
    nb = _N // _MMBLK

    def body(d_ref, dis_ref, ic_ref):
        deg = d_ref[0, :, 0:1] + d_ref[1, :, 0:1]
        dis = jnp.where(deg > 0, jax.lax.rsqrt(jnp.maximum(deg, 1e-12)), 0.0)
        dis_ref[...] = dis
        ic_ref[...] = 1.0 / jnp.maximum(deg, 1.0)

    return pl.pallas_call(
        body,
        grid=(nb,),
        in_specs=[pl.BlockSpec((2, _MMBLK, 16), lambda i: (0, i, 0))],
        out_specs=[
            pl.BlockSpec((_MMBLK, 1), lambda i: (i, 0)),
            pl.BlockSpec((_MMBLK, 1), lambda i: (i, 0)),
        ],
        out_shape=[
            jax.ShapeDtypeStruct((_N, 1), jnp.float32),
            jax.ShapeDtypeStruct((_N, 1), jnp.float32),
        ],
    )(degp)


def _tc_final(n4p, z4p, u_tag, p1, p2, p3, v_sage, hs_p, invcnt, scal, y, mask):
    nb = _N // _MMBLK

    def body(n4, z4, ut, p1r, p2r, p3r, vs, hs, ic, sc, y_ref, m_ref, o_ref):
        b4 = sc[0, 0]
        btag = sc[1, 0]
        bl3 = sc[2, 0]
        w0, w1, w2, blin = sc[3, 0], sc[4, 0], sc[5, 0], sc[6, 0]
        x1 = (n4[0, :, 0:1] + n4[1, :, 0:1]) / (z4[0, :, 0:1] + z4[1, :, 0:1]) + b4
        x2 = ut[:, 3:4] + p1r[:, 0:1] + p2r[:, 1:2] + p3r[:, 2:3] + btag
        x3 = (hs[0, :, 0:1] + hs[1, :, 0:1]) * ic[...] + bl3 + vs[:, 1:2]
        out = jnp.maximum(x1 * w0 + x2 * w1 + x3 * w2 + blin, 0.0)
        x_i = jnp.where(m_ref[...] != 0, out / 0.05, 0.0)
        o_ref[...] = jnp.where(y_ref[...] == 0.0, x_i, out)

    blk2 = pl.BlockSpec((2, _MMBLK, 16), lambda i: (0, i, 0))
    blk16 = pl.BlockSpec((_MMBLK, 16), lambda i: (i, 0))
    blk1 = pl.BlockSpec((_MMBLK, 1), lambda i: (i, 0))
    return pl.pallas_call(
        body,
        grid=(nb,),
        in_specs=[blk2, blk2, blk16, blk16, blk16, blk16, blk16, blk2,
                  blk1, pl.BlockSpec((7, 1), lambda i: (0, 0)), blk1, blk1],
        out_specs=blk1,
        out_shape=jax.ShapeDtypeStruct((_N, 1), jnp.float32),
    )(n4p, z4p, u_tag, p1, p2, p3, v_sage, hs_p, invcnt, scal, y, mask)


def _pad1(a, ep, fill=0):
    return jnp.concatenate(
        [a.astype(jnp.int32),
         jnp.full((ep - a.shape[0],), fill, jnp.int32)])


def _pad_trash(a, ep):
    npad = ep - a.shape[0]
    fill = _TRASH + (jnp.arange(npad, dtype=jnp.int32) % (_NACC - _N))
    return jnp.concatenate([a.astype(jnp.int32), fill])


def kernel(x, edge_index, y, params):
    n = _N
    src = edge_index[0].astype(jnp.int32)
    dst = edge_index[1].astype(jnp.int32)
    e = src.shape[0]
    loop = jnp.arange(n, dtype=jnp.int32)
    gran = _NC * _NS * _C

    ep1 = ((e + gran - 1) // gran) * gran
    ep2 = ((e + n + gran - 1) // gran) * gran
    epr1, epr2 = ep1 // _C, ep2 // _C

    srcp1 = _pad1(src, ep1)
    dstp1 = _pad_trash(dst, ep1)
    src2_1 = jnp.stack([srcp1, srcp1 + n])

    src_sl = jnp.concatenate([src, loop])
    dst_sl = jnp.concatenate([dst, loop])
    srcp2 = _pad1(src_sl, ep2)
    dstp2 = _pad_trash(dst_sl, ep2)
    dstg2 = _pad1(dst_sl, ep2)
    src2_2 = jnp.stack([srcp2, srcp2 + n])
    dst2_2 = jnp.stack([dstg2, dstg2 + n])

    xh = jnp.transpose(x.reshape(n, 2, 128), (1, 0, 2)).reshape(2 * n, 128)

    ones16 = jnp.ones((ep1, 16), jnp.float32)
    degp = _sc_scatter16(ones16, dstp1)
    dis, invcnt = _tc_prep(degp)
    dis2 = jnp.concatenate([dis, dis], axis=0)
    dis2sq = dis2 * dis2
    invcnt2 = jnp.concatenate([invcnt, invcnt], axis=0)

    x1h = xh
    for p in params['gat'][:3]:
        xl = _mm_h(x1h, p['Wl'])
        xr = _mm_h(x1h, p['Wr'])
        gl, gr = _sc_gather_pair(xl, xr, src2_2, dst2_2, width=128)
        s, gmax = _tc_score128(gl, gr, p['att'])
        e16 = _tc_exp16(s, gmax)
        zp = _sc_scatter16(e16, dstp2)
        z = zp[0, :, 0:1] + zp[1, :, 0:1]
        z2 = jnp.concatenate([z, z], axis=0)
        numer = _sc_hop(xl, src2_2, dstp2, w2d=e16, width=128).reshape(2 * n, 128)
        x1h = _tc_rowscale(numer, z2, bias=p['b'], relu=True, div=True)

    p4 = params['gat'][3]
    w4 = jnp.concatenate(
        [p4['Wl'], p4['Wr'], jnp.zeros((256, 14), jnp.float32)], axis=1)
    t4 = _mm_thin(x1h, w4)
    g1, g2 = _sc_gather_pair(t4, t4, srcp2, dstg2, width=16, edge_split=True)
    s4, gmax4 = _tc_score16(g1, g2, p4['att'])
    e4 = _tc_exp16(s4, gmax4)
    z4p = _sc_scatter16(e4, dstp2)
    n4p = _sc_hop(t4, srcp2, dstp2, w2d=e4, width=16, edge_split=True)

    x2h = xh
    for li, p in enumerate(params['tag'][:2]):
        out = _mm_h(x2h, p['Ws'][0])
        hs = _tc_rowscale(x2h, dis2)
        for kk in range(1, 4):
            raw = _sc_hop(hs, src2_1, dstp1, width=128).reshape(2 * n, 128)
            last = kk == 3
            out = _mm_h(raw, p['Ws'][kk], row_scale=dis2, acc=out,
                        bias=p['b'] if last else None, relu=last)
            if not last:
                hs = _tc_rowscale(raw, dis2sq)
        x2h = out

    p3t = params['tag'][2]
    w16t = jnp.concatenate(
        [p3t['Ws'][1], p3t['Ws'][2], p3t['Ws'][3], p3t['Ws'][0],
         jnp.zeros((256, 12), jnp.float32)], axis=1)
    u_tag = _mm_thin(x2h, w16t)
    q = _tc_scale16(u_tag, dis)
    h1 = _sc_hop(q, srcp1, dstp1, width=16, edge_split=True)
    pp1 = _tc_scale16p(h1, dis)
    q = _tc_scale16(pp1, dis)
    h2 = _sc_hop(q, srcp1, dstp1, width=16, edge_split=True)
    pp2 = _tc_scale16p(h2, dis)
    q = _tc_scale16(pp2, dis)
    h3 = _sc_hop(q, srcp1, dstp1, width=16, edge_split=True)
    pp3 = _tc_scale16p(h3, dis)

    x3h = xh
    for p in params['sage'][:2]:
        raw = _sc_hop(x3h, src2_1, dstp1, width=128).reshape(2 * n, 128)
        out = _mm_h(raw, p['Wl'], row_scale=invcnt2, bias=p['bl'])
        x3h = _mm_h(x3h, p['Wr'], acc=out, relu=True)

    p3s = params['sage'][2]
    w16s = jnp.concatenate(
        [p3s['Wl'], p3s['Wr'], jnp.zeros((256, 14), jnp.float32)], axis=1)
    v_sage = _mm_thin(x3h, w16s)
    hs_p = _sc_hop(v_sage, srcp1, dstp1, width=16, edge_split=True)

    scal = jnp.stack([
        p4['b'][0], p3t['b'][0], p3s['bl'][0],
        params['lin']['W'][0, 0], params['lin']['W'][1, 0],
        params['lin']['W'][2, 0], params['lin']['b'][0],
    ]).reshape(7, 1)
    mask = jax.random.bernoulli(jax.random.key(42), 0.05, (n, 1)).astype(jnp.float32)
    return _tc_final(n4p, z4p, u_tag, pp1, pp2, pp3, v_sage, hs_p, invcnt,
                     scal, y, mask)
